# Initial kernel scaffold; baseline (speedup 1.0000x reference)
import jax
import jax.numpy as jnp
from jax.experimental import pallas as pl


def kernel(x, edge_index, lin1_w, lin1_b, gat_W, att_src, att_dst, gat_b, lstm_Wih, lstm_Whh, lin2_w, lin2_b):
    raise NotImplementedError("write your pallas kernel here")


# speedup vs baseline: 19.5246x; 19.5246x over previous
import functools

import jax
import jax.numpy as jnp
from jax import lax
from jax.experimental import pallas as pl
from jax.experimental.pallas import tpu as pltpu
from jax.experimental.pallas import tpu_sc as plsc

N = 10000
E = 320000
D = 128
NL = 3
HID = 128
RES_W = 0.1
F32 = jnp.float32

NC = 2
NS = 16
NW = NC * NS
EPT = E // NW

NP = 30720
SLC = NP // NS
KB = 2000
NCHB = EPT // KB
KC = 80
NCHC = EPT // KC
NT = N // NS
ZR = 25
SR = 8


def _sc_denom(src, dst, asf, adf):
    mesh = plsc.VectorSubcoreMesh(core_axis_name="c", subcore_axis_name="s")

    @functools.partial(
        pl.kernel,
        out_type=(
            jax.ShapeDtypeStruct((NC * NP,), F32),
            jax.ShapeDtypeStruct((NL * E,), F32),
        ),
        mesh=mesh,
        scratch_types=[
            pltpu.VMEM((NL * N,), F32),
            pltpu.VMEM((NL * N,), F32),
            pltpu.VMEM((NP,), F32),
            pltpu.VMEM((KB,), jnp.int32),
            pltpu.VMEM((KB,), jnp.int32),
            pltpu.VMEM((NL * KB,), F32),
            pltpu.VMEM((SLC,), F32),
            pltpu.VMEM((SLC,), F32),
            pltpu.VMEM_SHARED((SR * NP,), F32),
        ],
        compiler_params=pltpu.CompilerParams(needs_layout_passes=False),
    )
    def body(src_h, dst_h, as_h, ad_h, pden_h, ee_h,
             asl, adl, dnl, srcv, dstv, eev, red, tbuf, shd):
        c = lax.axis_index("c")
        s = lax.axis_index("s")
        wid = c * NS + s
        base = wid * EPT

        @pl.loop(0, NP // 16)
        def _zero(i):
            dnl[pl.ds(i * 16, 16)] = jnp.zeros((16,), F32)

        pltpu.sync_copy(as_h, asl)
        pltpu.sync_copy(ad_h, adl)

        for ch in range(NCHB):
            off = base + ch * KB
            pltpu.sync_copy(src_h.at[pl.ds(off, KB)], srcv)
            pltpu.sync_copy(dst_h.at[pl.ds(off, KB)], dstv)

            @pl.loop(0, KB // 16)
            def _edges(g):
                sv = srcv[pl.ds(g * 16, 16)]
                dv = dstv[pl.ds(g * 16, 16)]
                for l in range(NL):
                    a = plsc.load_gather(asl, [sv + l * N])
                    b = plsc.load_gather(adl, [dv + l * N])
                    e = a + b
                    e = jnp.maximum(e, 0.2 * e)
                    ee = jnp.exp(e)
                    eev[pl.ds(l * KB + g * 16, 16)] = ee
                    plsc.addupdate_scatter(dnl, [dv + l * N], ee)

            for l in range(NL):
                pltpu.sync_copy(eev.at[pl.ds(l * KB, KB)],
                                ee_h.at[pl.ds(l * E + off, KB)])

        for r in range(NS // SR):

            @pl.when(jnp.logical_and(s >= r * SR, s < (r + 1) * SR))
            def _stage():
                pltpu.sync_copy(dnl, shd.at[pl.ds((s - r * SR) * NP, NP)])

            plsc.subcore_barrier()
            for t in range(SR):
                pltpu.sync_copy(shd.at[pl.ds(t * NP + s * SLC, SLC)], tbuf)
                if r == 0 and t == 0:

                    @pl.loop(0, SLC // 16)
                    def _init(i):
                        ix = pl.ds(i * 16, 16)
                        red[ix] = tbuf[ix]

                else:

                    @pl.loop(0, SLC // 16)
                    def _acc(i):
                        ix = pl.ds(i * 16, 16)
                        red[ix] = red[ix] + tbuf[ix]

            plsc.subcore_barrier()

        pltpu.sync_copy(red, pden_h.at[pl.ds(c * NP + s * SLC, SLC)])

    return body(src, dst, asf, adf)


def _sc_accum(src, dst, ee, denp, x0):
    mesh = plsc.VectorSubcoreMesh(core_axis_name="c", subcore_axis_name="s")

    @functools.partial(
        pl.kernel,
        out_type=jax.ShapeDtypeStruct((NC, NL, N, D), F32),
        mesh=mesh,
        scratch_types=[
            pltpu.VMEM((N,), F32),
            pltpu.VMEM((KC,), jnp.int32),
            pltpu.VMEM((KC,), jnp.int32),
            pltpu.VMEM((KC,), F32),
            pltpu.VMEM((KC,), F32),
            pltpu.VMEM((KC, D), F32),
            pltpu.VMEM((ZR, D), F32),
            pltpu.VMEM_SHARED((N, D), F32),
            pltpu.SemaphoreType.DMA,
        ],
        compiler_params=pltpu.CompilerParams(needs_layout_passes=False),
    )
    def body(src_h, dst_h, ee_h, den_h, x0_h, accp_h,
             dloc, srcv, dstv, eevc, alph, rows, zbuf, acc, gsem):
        c = lax.axis_index("c")
        s = lax.axis_index("s")
        wid = c * NS + s
        base = wid * EPT

        @pl.loop(0, ZR)
        def _zrow(i):
            for j in range(D // 16):
                zbuf[i, pl.ds(j * 16, 16)] = jnp.zeros((16,), F32)

        for l in range(NL):
            pltpu.sync_copy(den_h.at[pl.ds(l * N, N)], dloc)
            for z in range(NT // ZR):
                pltpu.sync_copy(zbuf, acc.at[pl.ds(s * NT + z * ZR, ZR)])
            plsc.subcore_barrier()

            @pl.loop(0, NCHC)
            def _chunk(i):
                off = base + i * KC
                pltpu.sync_copy(src_h.at[pl.ds(off, KC)], srcv)
                pltpu.sync_copy(dst_h.at[pl.ds(off, KC)], dstv)
                pltpu.sync_copy(ee_h.at[pl.ds(l * E + off, KC)], eevc)

                @pl.loop(0, KC // 16)
                def _alpha(g):
                    gx = pl.ds(g * 16, 16)
                    dn = plsc.load_gather(dloc, [dstv[gx]])
                    alph[gx] = eevc[gx] / dn

                pltpu.async_copy(x0_h.at[srcv], rows, gsem).wait()

                @pl.loop(0, KC)
                def _scale(e):
                    sp = plsc.load_gather(
                        alph, [jnp.full((16,), 0, jnp.int32) + e])
                    for j in range(D // 16):
                        jx = pl.ds(j * 16, 16)
                        rows[e, jx] = rows[e, jx] * sp

                pltpu.sync_copy(rows, acc.at[dstv], add=True)

            plsc.subcore_barrier()

            @pl.when(s == 0)
            def _dump():
                pltpu.sync_copy(acc, accp_h.at[c, l])

            plsc.subcore_barrier()

    return body(src, dst, ee, denp, x0)


def _tc_prep(x, w1, b1, u128):
    R = 400
    grid = (N // R,)

    def body(x_ref, w_ref, b_ref, u_ref, x0_ref, meta_ref, sexp_ref):
        x0 = jnp.dot(x_ref[...], w_ref[...],
                     preferred_element_type=F32) + b_ref[0]
        m = jnp.dot(x0, u_ref[...], preferred_element_type=F32)
        x0_ref[...] = x0
        meta_ref[...] = m
        sexp_ref[...] = jnp.exp(jnp.maximum(m, 0.2 * m))

    return pl.pallas_call(
        body,
        grid=grid,
        in_specs=[
            pl.BlockSpec((R, D), lambda i: (i, 0)),
            pl.BlockSpec((D, D), lambda i: (0, 0)),
            pl.BlockSpec((1, D), lambda i: (0, 0)),
            pl.BlockSpec((D, D), lambda i: (0, 0)),
        ],
        out_specs=[
            pl.BlockSpec((R, D), lambda i: (i, 0)),
            pl.BlockSpec((R, D), lambda i: (i, 0)),
            pl.BlockSpec((R, D), lambda i: (i, 0)),
        ],
        out_shape=[
            jax.ShapeDtypeStruct((N, D), F32),
            jax.ShapeDtypeStruct((N, D), F32),
            jax.ShapeDtypeStruct((N, D), F32),
        ],
    )(x, w1, b1, u128)


def _tc_final(accp, x0, selfw, gat_W, gat_b, wih_t, whh_t, w2, b2):
    R = 400
    grid = (N // R,)

    def body(a_ref, x0_ref, sw_ref, gw_ref, gb_ref, wih_ref, whh_ref,
             w2_ref, b2_ref, out_ref):
        x0 = x0_ref[...]
        sw = sw_ref[...]
        hs = []
        for l in range(NL):
            acc = a_ref[0, l] + a_ref[1, l]
            msg = acc + sw[:, l:l + 1] * x0
            h_l = jnp.tanh(
                jnp.dot(msg, gw_ref[l], preferred_element_type=F32)
                + gb_ref[l, 0])
            hs.append(h_l)
        h = jnp.zeros((R, HID), F32)
        cc = jnp.zeros((R, HID), F32)
        xx = x0
        for l in range(NL):
            cat = jnp.concatenate([hs[l], xx], axis=-1)
            g = (jnp.dot(cat, wih_ref[l], preferred_element_type=F32)
                 + jnp.dot(h, whh_ref[l], preferred_element_type=F32))
            gi = jax.nn.sigmoid(g[:, 0:HID])
            gf = jax.nn.sigmoid(g[:, HID:2 * HID])
            gg = jnp.tanh(g[:, 2 * HID:3 * HID])
            go = jax.nn.sigmoid(g[:, 3 * HID:4 * HID])
            cc = gf * cc + gi * gg
            h = go * jnp.tanh(cc)
            xx = h + RES_W * x0
        o = jnp.dot(xx, w2_ref[...], preferred_element_type=F32) + b2_ref[0]
        m = jnp.max(o, axis=-1, keepdims=True)
        lse = jnp.log(jnp.sum(jnp.exp(o - m), axis=-1, keepdims=True))
        out_ref[...] = o - m - lse

    return pl.pallas_call(
        body,
        grid=grid,
        in_specs=[
            pl.BlockSpec((NC, NL, R, D), lambda i: (0, 0, i, 0)),
            pl.BlockSpec((R, D), lambda i: (i, 0)),
            pl.BlockSpec((R, D), lambda i: (i, 0)),
            pl.BlockSpec((NL, D, D), lambda i: (0, 0, 0)),
            pl.BlockSpec((NL, 1, D), lambda i: (0, 0, 0)),
            pl.BlockSpec((NL, 2 * D, 4 * HID), lambda i: (0, 0, 0)),
            pl.BlockSpec((NL, HID, 4 * HID), lambda i: (0, 0, 0)),
            pl.BlockSpec((D, D), lambda i: (0, 0)),
            pl.BlockSpec((1, D), lambda i: (0, 0)),
        ],
        out_specs=pl.BlockSpec((R, D), lambda i: (i, 0)),
        out_shape=jax.ShapeDtypeStruct((N, D), F32),
    )(accp, x0, selfw, gat_W, gat_b, wih_t, whh_t, w2, b2)


def kernel(x, edge_index, lin1_w, lin1_b, gat_W, att_src, att_dst, gat_b,
           lstm_Wih, lstm_Whh, lin2_w, lin2_b):
    src = edge_index[0].astype(jnp.int32)
    dst = edge_index[1].astype(jnp.int32)

    u = jnp.einsum("lio,lo->li", gat_W, att_src)
    v = jnp.einsum("lio,lo->li", gat_W, att_dst)
    u128 = jnp.zeros((D, D), F32)
    u128 = u128.at[:, 0:NL].set(u.T)
    u128 = u128.at[:, NL:2 * NL].set(v.T)
    u128 = u128.at[:, 2 * NL:3 * NL].set(u.T + v.T)

    x0, meta, sexp = _tc_prep(x, lin1_w, lin1_b.reshape(1, D), u128)

    as_ = meta[:, 0:NL].T
    ad_ = meta[:, NL:2 * NL].T
    eeself = sexp[:, 2 * NL:3 * NL].T

    asf = as_.reshape(-1)
    adf = ad_.reshape(-1)

    pden, ee = _sc_denom(src, dst, asf, adf)
    pden = pden.reshape(NC, NP)

    denom = (pden[0, :NL * N] + pden[1, :NL * N]
             + eeself.reshape(-1) + 1e-16)
    denp = jnp.pad(denom, (0, NP - NL * N))

    alpha_self = eeself / denom.reshape(NL, N)
    selfw = jnp.zeros((N, D), F32).at[:, 0:NL].set(alpha_self.T)

    accp = _sc_accum(src, dst, ee, denp, x0)

    wih_t = jnp.transpose(lstm_Wih, (0, 2, 1))
    whh_t = jnp.transpose(lstm_Whh, (0, 2, 1))

    return _tc_final(accp, x0, selfw, gat_W,
                     gat_b.reshape(NL, 1, D), wih_t, whh_t,
                     lin2_w, lin2_b.reshape(1, D))


# speedup vs baseline: 32.5584x; 1.6676x over previous
import functools

import jax
import jax.numpy as jnp
from jax import lax
from jax.experimental import pallas as pl
from jax.experimental.pallas import tpu as pltpu
from jax.experimental.pallas import tpu_sc as plsc

N = 10000
E = 320000
D = 128
NL = 3
HID = 128
RES_W = 0.1
F32 = jnp.float32

NC = 2
NS = 16
NW = NC * NS
EPT = E // NW

NP = 30720
SLC = NP // NS
KB = 2000
NCHB = EPT // KB
KC = 80
NCHC = EPT // KC
NT = N // NS
ZR = 25
SR = 8


def _sc_denom(src, dst, asf, adf):
    mesh = plsc.VectorSubcoreMesh(core_axis_name="c", subcore_axis_name="s")

    @functools.partial(
        pl.kernel,
        out_type=(
            jax.ShapeDtypeStruct((NC * NP,), F32),
            jax.ShapeDtypeStruct((NL * E,), F32),
        ),
        mesh=mesh,
        scratch_types=[
            pltpu.VMEM((NL * N,), F32),
            pltpu.VMEM((NL * N,), F32),
            pltpu.VMEM((NP,), F32),
            pltpu.VMEM((KB,), jnp.int32),
            pltpu.VMEM((KB,), jnp.int32),
            pltpu.VMEM((NL * KB,), F32),
            pltpu.VMEM((SLC,), F32),
            pltpu.VMEM((SLC,), F32),
            pltpu.VMEM_SHARED((SR * NP,), F32),
        ],
        compiler_params=pltpu.CompilerParams(needs_layout_passes=False),
    )
    def body(src_h, dst_h, as_h, ad_h, pden_h, ee_h,
             asl, adl, dnl, srcv, dstv, eev, red, tbuf, shd):
        c = lax.axis_index("c")
        s = lax.axis_index("s")
        wid = c * NS + s
        base = wid * EPT

        @pl.loop(0, NP // 16)
        def _zero(i):
            dnl[pl.ds(i * 16, 16)] = jnp.zeros((16,), F32)

        pltpu.sync_copy(as_h, asl)
        pltpu.sync_copy(ad_h, adl)

        for ch in range(NCHB):
            off = base + ch * KB
            pltpu.sync_copy(src_h.at[pl.ds(off, KB)], srcv)
            pltpu.sync_copy(dst_h.at[pl.ds(off, KB)], dstv)

            @pl.loop(0, KB // 16)
            def _edges(g):
                sv = srcv[pl.ds(g * 16, 16)]
                dv = dstv[pl.ds(g * 16, 16)]
                for l in range(NL):
                    a = plsc.load_gather(asl, [sv + l * N])
                    b = plsc.load_gather(adl, [dv + l * N])
                    e = a + b
                    e = jnp.maximum(e, 0.2 * e)
                    ee = jnp.exp(e)
                    eev[pl.ds(l * KB + g * 16, 16)] = ee
                    plsc.addupdate_scatter(dnl, [dv + l * N], ee)

            for l in range(NL):
                pltpu.sync_copy(eev.at[pl.ds(l * KB, KB)],
                                ee_h.at[pl.ds(l * E + off, KB)])

        for r in range(NS // SR):

            @pl.when(jnp.logical_and(s >= r * SR, s < (r + 1) * SR))
            def _stage():
                pltpu.sync_copy(dnl, shd.at[pl.ds((s - r * SR) * NP, NP)])

            plsc.subcore_barrier()
            for t in range(SR):
                pltpu.sync_copy(shd.at[pl.ds(t * NP + s * SLC, SLC)], tbuf)
                if r == 0 and t == 0:

                    @pl.loop(0, SLC // 16)
                    def _init(i):
                        ix = pl.ds(i * 16, 16)
                        red[ix] = tbuf[ix]

                else:

                    @pl.loop(0, SLC // 16)
                    def _acc(i):
                        ix = pl.ds(i * 16, 16)
                        red[ix] = red[ix] + tbuf[ix]

            plsc.subcore_barrier()

        pltpu.sync_copy(red, pden_h.at[pl.ds(c * NP + s * SLC, SLC)])

    return body(src, dst, asf, adf)


def _sc_accum(src, dst, ee, denp, x0):
    mesh = plsc.VectorSubcoreMesh(core_axis_name="c", subcore_axis_name="s")

    @functools.partial(
        pl.kernel,
        out_type=jax.ShapeDtypeStruct((NC, NL, N, D), F32),
        mesh=mesh,
        scratch_types=[
            pltpu.VMEM((N,), F32),
            pltpu.VMEM((KC,), jnp.int32),
            pltpu.VMEM((KC,), jnp.int32),
            pltpu.VMEM((KC,), jnp.int32),
            pltpu.VMEM((KC,), jnp.int32),
            pltpu.VMEM((KC,), F32),
            pltpu.VMEM((KC,), F32),
            pltpu.VMEM((KC,), F32),
            pltpu.VMEM((KC,), F32),
            pltpu.VMEM((KC, D), F32),
            pltpu.VMEM((KC, D), F32),
            pltpu.VMEM((ZR, D), F32),
            pltpu.VMEM_SHARED((N, D), F32),
            pltpu.SemaphoreType.DMA,
            pltpu.SemaphoreType.DMA,
            pltpu.SemaphoreType.DMA,
            pltpu.SemaphoreType.DMA,
            pltpu.SemaphoreType.DMA,
            pltpu.SemaphoreType.DMA,
        ],
        compiler_params=pltpu.CompilerParams(needs_layout_passes=False),
    )
    def body(src_h, dst_h, ee_h, den_h, x0_h, accp_h,
             dloc, srcv0, srcv1, dstv0, dstv1, eevc0, eevc1, alph0, alph1,
             rows0, rows1, zbuf, acc, lsem0, lsem1, gsem0, gsem1,
             ssem0, ssem1):
        c = lax.axis_index("c")
        s = lax.axis_index("s")
        wid = c * NS + s
        base = wid * EPT
        svs = (srcv0, srcv1)
        dvs = (dstv0, dstv1)
        evs = (eevc0, eevc1)
        als = (alph0, alph1)
        rws = (rows0, rows1)
        lsems = (lsem0, lsem1)
        gsems = (gsem0, gsem1)
        ssems = (ssem0, ssem1)

        @pl.loop(0, ZR)
        def _zrow(i):
            for j in range(D // 16):
                zbuf[i, pl.ds(j * 16, 16)] = jnp.zeros((16,), F32)

        for l in range(NL):

            def _off(i):
                return base + jnp.minimum(i, NCHC - 1) * KC

            def lin_descs(b, i):
                off = _off(i)
                return (
                    pltpu.make_async_copy(src_h.at[pl.ds(off, KC)],
                                          svs[b], lsems[b]),
                    pltpu.make_async_copy(dst_h.at[pl.ds(off, KC)],
                                          dvs[b], lsems[b]),
                    pltpu.make_async_copy(ee_h.at[pl.ds(l * E + off, KC)],
                                          evs[b], lsems[b]),
                )

            def issue_lin(b, i):
                for d in lin_descs(b, i):
                    d.start()

            def wait_lin(b, i):
                for d in lin_descs(b, i):
                    d.wait()

            def issue_gather(b):
                pltpu.async_copy(x0_h.at[svs[b]], rws[b], gsems[b])

            def wait_gather(b):
                pltpu.make_async_copy(x0_h.at[svs[b]], rws[b],
                                      gsems[b]).wait()

            def compute_alpha(b):
                @pl.loop(0, KC // 16)
                def _alpha(g):
                    gx = pl.ds(g * 16, 16)
                    dn = plsc.load_gather(dloc, [dvs[b][gx]])
                    als[b][gx] = evs[b][gx] / dn

            def scale(b):
                @pl.loop(0, KC)
                def _scale(e):
                    sp = plsc.load_gather(
                        als[b], [jnp.full((16,), 0, jnp.int32) + e])
                    for j in range(D // 16):
                        jx = pl.ds(j * 16, 16)
                        rws[b][e, jx] = rws[b][e, jx] * sp

            def issue_scat(b):
                pltpu.async_copy(rws[b], acc.at[dvs[b]], ssems[b], add=True)

            def wait_scat(b):
                pltpu.make_async_copy(rws[b], acc.at[dvs[b]],
                                      ssems[b]).wait()

            pltpu.sync_copy(den_h.at[pl.ds(l * N, N)], dloc)
            for z in range(NT // ZR):
                pltpu.sync_copy(zbuf, acc.at[pl.ds(s * NT + z * ZR, ZR)])
            plsc.subcore_barrier()

            issue_lin(0, 0)
            wait_lin(0, 0)
            issue_gather(0)
            compute_alpha(0)
            issue_lin(1, 1)

            @pl.loop(0, (NCHC - 1) // 2)
            def _pair(t):
                i = t * 2
                wait_gather(0)
                scale(0)
                issue_scat(0)
                wait_lin(1, i + 1)
                issue_gather(1)
                compute_alpha(1)
                wait_scat(0)
                issue_lin(0, i + 2)
                wait_gather(1)
                scale(1)
                issue_scat(1)
                wait_lin(0, i + 2)
                issue_gather(0)
                compute_alpha(0)
                wait_scat(1)
                issue_lin(1, i + 3)

            wait_gather(0)
            scale(0)
            pltpu.sync_copy(rws[0], acc.at[dvs[0]], add=True)
            wait_lin(1, NCHC)

            plsc.subcore_barrier()

            @pl.when(s == 0)
            def _dump():
                pltpu.sync_copy(acc, accp_h.at[c, l])

            plsc.subcore_barrier()

    return body(src, dst, ee, denp, x0)


def _tc_prep(x, w1, b1, u128):
    R = 400
    grid = (N // R,)

    def body(x_ref, w_ref, b_ref, u_ref, x0_ref, meta_ref, sexp_ref):
        x0 = jnp.dot(x_ref[...], w_ref[...],
                     preferred_element_type=F32) + b_ref[0]
        m = jnp.dot(x0, u_ref[...], preferred_element_type=F32)
        x0_ref[...] = x0
        meta_ref[...] = m
        sexp_ref[...] = jnp.exp(jnp.maximum(m, 0.2 * m))

    return pl.pallas_call(
        body,
        grid=grid,
        in_specs=[
            pl.BlockSpec((R, D), lambda i: (i, 0)),
            pl.BlockSpec((D, D), lambda i: (0, 0)),
            pl.BlockSpec((1, D), lambda i: (0, 0)),
            pl.BlockSpec((D, D), lambda i: (0, 0)),
        ],
        out_specs=[
            pl.BlockSpec((R, D), lambda i: (i, 0)),
            pl.BlockSpec((R, D), lambda i: (i, 0)),
            pl.BlockSpec((R, D), lambda i: (i, 0)),
        ],
        out_shape=[
            jax.ShapeDtypeStruct((N, D), F32),
            jax.ShapeDtypeStruct((N, D), F32),
            jax.ShapeDtypeStruct((N, D), F32),
        ],
    )(x, w1, b1, u128)


def _tc_final(accp, x0, selfw, gat_W, gat_b, wih_t, whh_t, w2, b2):
    R = 400
    grid = (N // R,)

    def body(a_ref, x0_ref, sw_ref, gw_ref, gb_ref, wih_ref, whh_ref,
             w2_ref, b2_ref, out_ref):
        x0 = x0_ref[...]
        sw = sw_ref[...]
        hs = []
        for l in range(NL):
            acc = a_ref[0, l] + a_ref[1, l]
            msg = acc + sw[:, l:l + 1] * x0
            h_l = jnp.tanh(
                jnp.dot(msg, gw_ref[l], preferred_element_type=F32)
                + gb_ref[l, 0])
            hs.append(h_l)
        h = jnp.zeros((R, HID), F32)
        cc = jnp.zeros((R, HID), F32)
        xx = x0
        for l in range(NL):
            cat = jnp.concatenate([hs[l], xx], axis=-1)
            g = (jnp.dot(cat, wih_ref[l], preferred_element_type=F32)
                 + jnp.dot(h, whh_ref[l], preferred_element_type=F32))
            gi = jax.nn.sigmoid(g[:, 0:HID])
            gf = jax.nn.sigmoid(g[:, HID:2 * HID])
            gg = jnp.tanh(g[:, 2 * HID:3 * HID])
            go = jax.nn.sigmoid(g[:, 3 * HID:4 * HID])
            cc = gf * cc + gi * gg
            h = go * jnp.tanh(cc)
            xx = h + RES_W * x0
        o = jnp.dot(xx, w2_ref[...], preferred_element_type=F32) + b2_ref[0]
        m = jnp.max(o, axis=-1, keepdims=True)
        lse = jnp.log(jnp.sum(jnp.exp(o - m), axis=-1, keepdims=True))
        out_ref[...] = o - m - lse

    return pl.pallas_call(
        body,
        grid=grid,
        in_specs=[
            pl.BlockSpec((NC, NL, R, D), lambda i: (0, 0, i, 0)),
            pl.BlockSpec((R, D), lambda i: (i, 0)),
            pl.BlockSpec((R, D), lambda i: (i, 0)),
            pl.BlockSpec((NL, D, D), lambda i: (0, 0, 0)),
            pl.BlockSpec((NL, 1, D), lambda i: (0, 0, 0)),
            pl.BlockSpec((NL, 2 * D, 4 * HID), lambda i: (0, 0, 0)),
            pl.BlockSpec((NL, HID, 4 * HID), lambda i: (0, 0, 0)),
            pl.BlockSpec((D, D), lambda i: (0, 0)),
            pl.BlockSpec((1, D), lambda i: (0, 0)),
        ],
        out_specs=pl.BlockSpec((R, D), lambda i: (i, 0)),
        out_shape=jax.ShapeDtypeStruct((N, D), F32),
    )(accp, x0, selfw, gat_W, gat_b, wih_t, whh_t, w2, b2)


def kernel(x, edge_index, lin1_w, lin1_b, gat_W, att_src, att_dst, gat_b,
           lstm_Wih, lstm_Whh, lin2_w, lin2_b):
    src = edge_index[0].astype(jnp.int32)
    dst = edge_index[1].astype(jnp.int32)

    u = jnp.einsum("lio,lo->li", gat_W, att_src)
    v = jnp.einsum("lio,lo->li", gat_W, att_dst)
    u128 = jnp.zeros((D, D), F32)
    u128 = u128.at[:, 0:NL].set(u.T)
    u128 = u128.at[:, NL:2 * NL].set(v.T)
    u128 = u128.at[:, 2 * NL:3 * NL].set(u.T + v.T)

    x0, meta, sexp = _tc_prep(x, lin1_w, lin1_b.reshape(1, D), u128)

    as_ = meta[:, 0:NL].T
    ad_ = meta[:, NL:2 * NL].T
    eeself = sexp[:, 2 * NL:3 * NL].T

    asf = as_.reshape(-1)
    adf = ad_.reshape(-1)

    pden, ee = _sc_denom(src, dst, asf, adf)
    pden = pden.reshape(NC, NP)

    denom = (pden[0, :NL * N] + pden[1, :NL * N]
             + eeself.reshape(-1) + 1e-16)
    denp = jnp.pad(denom, (0, NP - NL * N))

    alpha_self = eeself / denom.reshape(NL, N)
    selfw = jnp.zeros((N, D), F32).at[:, 0:NL].set(alpha_self.T)

    accp = _sc_accum(src, dst, ee, denp, x0)

    wih_t = jnp.transpose(lstm_Wih, (0, 2, 1))
    whh_t = jnp.transpose(lstm_Whh, (0, 2, 1))

    return _tc_final(accp, x0, selfw, gat_W,
                     gat_b.reshape(NL, 1, D), wih_t, whh_t,
                     lin2_w, lin2_b.reshape(1, D))


# speedup vs baseline: 34.3944x; 1.0564x over previous
import functools

import jax
import jax.numpy as jnp
from jax import lax
from jax.experimental import pallas as pl
from jax.experimental.pallas import tpu as pltpu
from jax.experimental.pallas import tpu_sc as plsc

N = 10000
E = 320000
D = 128
NL = 3
HID = 128
RES_W = 0.1
F32 = jnp.float32

NC = 2
NS = 16
NW = NC * NS
EPT = E // NW

NP = 30720
SLC = NP // NS
KB = 2000
NCHB = EPT // KB
KC = 80
NCHC = EPT // KC
NT = N // NS
SR = 8


def _sc_denom(src, dst, asf, adf):
    mesh = plsc.VectorSubcoreMesh(core_axis_name="c", subcore_axis_name="s")

    @functools.partial(
        pl.kernel,
        out_type=(
            jax.ShapeDtypeStruct((NC * NP,), F32),
            jax.ShapeDtypeStruct((NL * E,), F32),
        ),
        mesh=mesh,
        scratch_types=[
            pltpu.VMEM((NL * N,), F32),
            pltpu.VMEM((NL * N,), F32),
            pltpu.VMEM((NP,), F32),
            pltpu.VMEM((KB,), jnp.int32),
            pltpu.VMEM((KB,), jnp.int32),
            pltpu.VMEM((NL * KB,), F32),
            pltpu.VMEM((SLC,), F32),
            pltpu.VMEM((SLC,), F32),
            pltpu.VMEM_SHARED((SR * NP,), F32),
        ],
        compiler_params=pltpu.CompilerParams(needs_layout_passes=False),
    )
    def body(src_h, dst_h, as_h, ad_h, pden_h, ee_h,
             asl, adl, dnl, srcv, dstv, eev, red, tbuf, shd):
        c = lax.axis_index("c")
        s = lax.axis_index("s")
        wid = c * NS + s
        base = wid * EPT

        @pl.loop(0, NP // 16)
        def _zero(i):
            dnl[pl.ds(i * 16, 16)] = jnp.zeros((16,), F32)

        pltpu.sync_copy(as_h, asl)
        pltpu.sync_copy(ad_h, adl)

        for ch in range(NCHB):
            off = base + ch * KB
            pltpu.sync_copy(src_h.at[pl.ds(off, KB)], srcv)
            pltpu.sync_copy(dst_h.at[pl.ds(off, KB)], dstv)

            @pl.loop(0, KB // 16)
            def _edges(g):
                sv = srcv[pl.ds(g * 16, 16)]
                dv = dstv[pl.ds(g * 16, 16)]
                for l in range(NL):
                    a = plsc.load_gather(asl, [sv + l * N])
                    b = plsc.load_gather(adl, [dv + l * N])
                    e = a + b
                    e = jnp.maximum(e, 0.2 * e)
                    ee = jnp.exp(e)
                    eev[pl.ds(l * KB + g * 16, 16)] = ee
                    plsc.addupdate_scatter(dnl, [dv + l * N], ee)

            for l in range(NL):
                pltpu.sync_copy(eev.at[pl.ds(l * KB, KB)],
                                ee_h.at[pl.ds(l * E + off, KB)])

        for r in range(NS // SR):

            @pl.when(jnp.logical_and(s >= r * SR, s < (r + 1) * SR))
            def _stage():
                pltpu.sync_copy(dnl, shd.at[pl.ds((s - r * SR) * NP, NP)])

            plsc.subcore_barrier()
            for t in range(SR):
                pltpu.sync_copy(shd.at[pl.ds(t * NP + s * SLC, SLC)], tbuf)
                if r == 0 and t == 0:

                    @pl.loop(0, SLC // 16)
                    def _init(i):
                        ix = pl.ds(i * 16, 16)
                        red[ix] = tbuf[ix]

                else:

                    @pl.loop(0, SLC // 16)
                    def _acc(i):
                        ix = pl.ds(i * 16, 16)
                        red[ix] = red[ix] + tbuf[ix]

            plsc.subcore_barrier()

        pltpu.sync_copy(red, pden_h.at[pl.ds(c * NP + s * SLC, SLC)])

    return body(src, dst, asf, adf)


def _sc_accum(src, dst, ee, denp, x0, zrows):
    mesh = plsc.VectorSubcoreMesh(core_axis_name="c", subcore_axis_name="s")

    @functools.partial(
        pl.kernel,
        out_type=jax.ShapeDtypeStruct((NC, NL, N, D), F32),
        mesh=mesh,
        scratch_types=[
            pltpu.VMEM((N,), F32),
            pltpu.VMEM((KC,), jnp.int32),
            pltpu.VMEM((KC,), jnp.int32),
            pltpu.VMEM((KC,), jnp.int32),
            pltpu.VMEM((KC,), jnp.int32),
            pltpu.VMEM((KC,), F32),
            pltpu.VMEM((KC,), F32),
            pltpu.VMEM((KC,), F32),
            pltpu.VMEM((KC,), F32),
            pltpu.VMEM((KC, D), F32),
            pltpu.VMEM((KC, D), F32),
            pltpu.VMEM_SHARED((N, D), F32),
            pltpu.SemaphoreType.DMA,
            pltpu.SemaphoreType.DMA,
            pltpu.SemaphoreType.DMA,
            pltpu.SemaphoreType.DMA,
            pltpu.SemaphoreType.DMA,
            pltpu.SemaphoreType.DMA,
        ],
        compiler_params=pltpu.CompilerParams(needs_layout_passes=False),
    )
    def body(src_h, dst_h, ee_h, den_h, x0_h, z_h, accp_h,
             dloc, srcv0, srcv1, dstv0, dstv1, eevc0, eevc1, alph0, alph1,
             rows0, rows1, acc, lsem0, lsem1, gsem0, gsem1,
             ssem0, ssem1):
        c = lax.axis_index("c")
        s = lax.axis_index("s")
        wid = c * NS + s
        base = wid * EPT
        svs = (srcv0, srcv1)
        dvs = (dstv0, dstv1)
        evs = (eevc0, eevc1)
        als = (alph0, alph1)
        rws = (rows0, rows1)
        lsems = (lsem0, lsem1)
        gsems = (gsem0, gsem1)
        ssems = (ssem0, ssem1)

        for l in range(NL):

            def _off(i):
                return base + jnp.minimum(i, NCHC - 1) * KC

            def lin_descs(b, i):
                off = _off(i)
                return (
                    pltpu.make_async_copy(src_h.at[pl.ds(off, KC)],
                                          svs[b], lsems[b]),
                    pltpu.make_async_copy(dst_h.at[pl.ds(off, KC)],
                                          dvs[b], lsems[b]),
                    pltpu.make_async_copy(ee_h.at[pl.ds(l * E + off, KC)],
                                          evs[b], lsems[b]),
                )

            def issue_lin(b, i):
                for d in lin_descs(b, i):
                    d.start()

            def wait_lin(b, i):
                for d in lin_descs(b, i):
                    d.wait()

            def issue_gather(b):
                pltpu.async_copy(x0_h.at[svs[b]], rws[b], gsems[b])

            def wait_gather(b):
                pltpu.make_async_copy(x0_h.at[svs[b]], rws[b],
                                      gsems[b]).wait()

            def compute_alpha(b):
                @pl.loop(0, KC // 16)
                def _alpha(g):
                    gx = pl.ds(g * 16, 16)
                    dn = plsc.load_gather(dloc, [dvs[b][gx]])
                    als[b][gx] = evs[b][gx] / dn

            def scale(b):
                @pl.loop(0, KC)
                def _scale(e):
                    sp = plsc.load_gather(
                        als[b], [jnp.full((16,), 0, jnp.int32) + e])
                    for j in range(D // 16):
                        jx = pl.ds(j * 16, 16)
                        rws[b][e, jx] = rws[b][e, jx] * sp

            def issue_scat(b):
                pltpu.async_copy(rws[b], acc.at[dvs[b]], ssems[b], add=True)

            def wait_scat(b):
                pltpu.make_async_copy(rws[b], acc.at[dvs[b]],
                                      ssems[b]).wait()

            pltpu.sync_copy(den_h.at[pl.ds(l * N, N)], dloc)
            pltpu.sync_copy(z_h.at[pl.ds(0, 624)],
                            acc.at[pl.ds(s * 624, 624)])

            @pl.when(s == NS - 1)
            def _ztail():
                pltpu.sync_copy(z_h.at[pl.ds(0, 16)],
                                acc.at[pl.ds(624 * NS, 16)])

            plsc.subcore_barrier()

            issue_lin(0, 0)
            wait_lin(0, 0)
            issue_gather(0)
            compute_alpha(0)
            issue_lin(1, 1)

            @pl.loop(0, (NCHC - 1) // 2)
            def _pair(t):
                i = t * 2
                wait_gather(0)
                wait_lin(1, i + 1)
                issue_gather(1)
                scale(0)
                issue_scat(0)
                compute_alpha(1)
                wait_scat(0)
                issue_lin(0, i + 2)
                wait_gather(1)
                wait_lin(0, i + 2)
                issue_gather(0)
                scale(1)
                issue_scat(1)
                compute_alpha(0)
                wait_scat(1)
                issue_lin(1, i + 3)

            wait_gather(0)
            scale(0)
            pltpu.sync_copy(rws[0], acc.at[dvs[0]], add=True)
            wait_lin(1, NCHC)

            plsc.subcore_barrier()

            pltpu.sync_copy(acc.at[pl.ds(s * 624, 624)],
                            accp_h.at[c, l, pl.ds(s * 624, 624)])

            @pl.when(s == NS - 1)
            def _dtail():
                pltpu.sync_copy(acc.at[pl.ds(624 * NS, 16)],
                                accp_h.at[c, l, pl.ds(624 * NS, 16)])

            plsc.subcore_barrier()

    return body(src, dst, ee, denp, x0, zrows)


def _tc_prep(x, w1, b1, u128):
    R = 400
    grid = (N // R,)

    def body(x_ref, w_ref, b_ref, u_ref, x0_ref, meta_ref, sexp_ref):
        x0 = jnp.dot(x_ref[...], w_ref[...],
                     preferred_element_type=F32) + b_ref[0]
        m = jnp.dot(x0, u_ref[...], preferred_element_type=F32)
        x0_ref[...] = x0
        meta_ref[...] = m
        sexp_ref[...] = jnp.exp(jnp.maximum(m, 0.2 * m))

    return pl.pallas_call(
        body,
        grid=grid,
        in_specs=[
            pl.BlockSpec((R, D), lambda i: (i, 0)),
            pl.BlockSpec((D, D), lambda i: (0, 0)),
            pl.BlockSpec((1, D), lambda i: (0, 0)),
            pl.BlockSpec((D, D), lambda i: (0, 0)),
        ],
        out_specs=[
            pl.BlockSpec((R, D), lambda i: (i, 0)),
            pl.BlockSpec((R, D), lambda i: (i, 0)),
            pl.BlockSpec((R, D), lambda i: (i, 0)),
        ],
        out_shape=[
            jax.ShapeDtypeStruct((N, D), F32),
            jax.ShapeDtypeStruct((N, D), F32),
            jax.ShapeDtypeStruct((N, D), F32),
        ],
    )(x, w1, b1, u128)


def _tc_final(accp, x0, selfw, gat_W, gat_b, wih_t, whh_t, w2, b2):
    R = 400
    grid = (N // R,)

    def body(a_ref, x0_ref, sw_ref, gw_ref, gb_ref, wih_ref, whh_ref,
             w2_ref, b2_ref, out_ref):
        x0 = x0_ref[...]
        sw = sw_ref[...]
        hs = []
        for l in range(NL):
            acc = a_ref[0, l] + a_ref[1, l]
            msg = acc + sw[:, l:l + 1] * x0
            h_l = jnp.tanh(
                jnp.dot(msg, gw_ref[l], preferred_element_type=F32)
                + gb_ref[l, 0])
            hs.append(h_l)
        h = jnp.zeros((R, HID), F32)
        cc = jnp.zeros((R, HID), F32)
        xx = x0
        for l in range(NL):
            cat = jnp.concatenate([hs[l], xx], axis=-1)
            g = (jnp.dot(cat, wih_ref[l], preferred_element_type=F32)
                 + jnp.dot(h, whh_ref[l], preferred_element_type=F32))
            gi = jax.nn.sigmoid(g[:, 0:HID])
            gf = jax.nn.sigmoid(g[:, HID:2 * HID])
            gg = jnp.tanh(g[:, 2 * HID:3 * HID])
            go = jax.nn.sigmoid(g[:, 3 * HID:4 * HID])
            cc = gf * cc + gi * gg
            h = go * jnp.tanh(cc)
            xx = h + RES_W * x0
        o = jnp.dot(xx, w2_ref[...], preferred_element_type=F32) + b2_ref[0]
        m = jnp.max(o, axis=-1, keepdims=True)
        lse = jnp.log(jnp.sum(jnp.exp(o - m), axis=-1, keepdims=True))
        out_ref[...] = o - m - lse

    return pl.pallas_call(
        body,
        grid=grid,
        in_specs=[
            pl.BlockSpec((NC, NL, R, D), lambda i: (0, 0, i, 0)),
            pl.BlockSpec((R, D), lambda i: (i, 0)),
            pl.BlockSpec((R, D), lambda i: (i, 0)),
            pl.BlockSpec((NL, D, D), lambda i: (0, 0, 0)),
            pl.BlockSpec((NL, 1, D), lambda i: (0, 0, 0)),
            pl.BlockSpec((NL, 2 * D, 4 * HID), lambda i: (0, 0, 0)),
            pl.BlockSpec((NL, HID, 4 * HID), lambda i: (0, 0, 0)),
            pl.BlockSpec((D, D), lambda i: (0, 0)),
            pl.BlockSpec((1, D), lambda i: (0, 0)),
        ],
        out_specs=pl.BlockSpec((R, D), lambda i: (i, 0)),
        out_shape=jax.ShapeDtypeStruct((N, D), F32),
    )(accp, x0, selfw, gat_W, gat_b, wih_t, whh_t, w2, b2)


def kernel(x, edge_index, lin1_w, lin1_b, gat_W, att_src, att_dst, gat_b,
           lstm_Wih, lstm_Whh, lin2_w, lin2_b):
    src = edge_index[0].astype(jnp.int32)
    dst = edge_index[1].astype(jnp.int32)

    u = jnp.einsum("lio,lo->li", gat_W, att_src)
    v = jnp.einsum("lio,lo->li", gat_W, att_dst)
    u128 = jnp.zeros((D, D), F32)
    u128 = u128.at[:, 0:NL].set(u.T)
    u128 = u128.at[:, NL:2 * NL].set(v.T)
    u128 = u128.at[:, 2 * NL:3 * NL].set(u.T + v.T)

    x0, meta, sexp = _tc_prep(x, lin1_w, lin1_b.reshape(1, D), u128)

    as_ = meta[:, 0:NL].T
    ad_ = meta[:, NL:2 * NL].T
    eeself = sexp[:, 2 * NL:3 * NL].T

    asf = as_.reshape(-1)
    adf = ad_.reshape(-1)

    pden, ee = _sc_denom(src, dst, asf, adf)
    pden = pden.reshape(NC, NP)

    denom = (pden[0, :NL * N] + pden[1, :NL * N]
             + eeself.reshape(-1) + 1e-16)
    denp = jnp.pad(denom, (0, NP - NL * N))

    alpha_self = eeself / denom.reshape(NL, N)
    selfw = jnp.zeros((N, D), F32).at[:, 0:NL].set(alpha_self.T)

    zrows = jnp.zeros((624, D), F32)
    accp = _sc_accum(src, dst, ee, denp, x0, zrows)

    wih_t = jnp.transpose(lstm_Wih, (0, 2, 1))
    whh_t = jnp.transpose(lstm_Whh, (0, 2, 1))

    return _tc_final(accp, x0, selfw, gat_W,
                     gat_b.reshape(NL, 1, D), wih_t, whh_t,
                     lin2_w, lin2_b.reshape(1, D))


# speedup vs baseline: 41.7760x; 1.2146x over previous
import functools

import jax
import jax.numpy as jnp
from jax import lax
from jax.experimental import pallas as pl
from jax.experimental.pallas import tpu as pltpu
from jax.experimental.pallas import tpu_sc as plsc

N = 10000
E = 320000
D = 128
NL = 3
HID = 128
RES_W = 0.1
F32 = jnp.float32

NC = 2
NS = 16
NW = NC * NS
EPT = E // NW

NP = 30720
SLC = NP // NS
KB = 2000
NCHB = EPT // KB
KC = 80
LCH = 400
KPL = LCH // KC
NLCH = EPT // LCH
NT = N // NS
SR = 8


def _sc_denom(src, dst, asf, adf):
    mesh = plsc.VectorSubcoreMesh(core_axis_name="c", subcore_axis_name="s")

    @functools.partial(
        pl.kernel,
        out_type=(
            jax.ShapeDtypeStruct((NC * NP,), F32),
            jax.ShapeDtypeStruct((NL * E,), F32),
        ),
        mesh=mesh,
        scratch_types=[
            pltpu.VMEM((NL * N,), F32),
            pltpu.VMEM((NL * N,), F32),
            pltpu.VMEM((NP,), F32),
            pltpu.VMEM((KB,), jnp.int32),
            pltpu.VMEM((KB,), jnp.int32),
            pltpu.VMEM((NL * KB,), F32),
            pltpu.VMEM((SLC,), F32),
            pltpu.VMEM((SLC,), F32),
            pltpu.VMEM_SHARED((SR * NP,), F32),
        ],
        compiler_params=pltpu.CompilerParams(needs_layout_passes=False),
    )
    def body(src_h, dst_h, as_h, ad_h, pden_h, ee_h,
             asl, adl, dnl, srcv, dstv, eev, red, tbuf, shd):
        c = lax.axis_index("c")
        s = lax.axis_index("s")
        wid = c * NS + s
        base = wid * EPT

        @pl.loop(0, NP // 16)
        def _zero(i):
            dnl[pl.ds(i * 16, 16)] = jnp.zeros((16,), F32)

        pltpu.sync_copy(as_h, asl)
        pltpu.sync_copy(ad_h, adl)

        for ch in range(NCHB):
            off = base + ch * KB
            pltpu.sync_copy(src_h.at[pl.ds(off, KB)], srcv)
            pltpu.sync_copy(dst_h.at[pl.ds(off, KB)], dstv)

            @pl.loop(0, KB // 16)
            def _edges(g):
                sv = srcv[pl.ds(g * 16, 16)]
                dv = dstv[pl.ds(g * 16, 16)]
                for l in range(NL):
                    a = plsc.load_gather(asl, [sv + l * N])
                    b = plsc.load_gather(adl, [dv + l * N])
                    e = a + b
                    e = jnp.maximum(e, 0.2 * e)
                    ee = jnp.exp(e)
                    eev[pl.ds(l * KB + g * 16, 16)] = ee
                    plsc.addupdate_scatter(dnl, [dv + l * N], ee)

            for l in range(NL):
                pltpu.sync_copy(eev.at[pl.ds(l * KB, KB)],
                                ee_h.at[pl.ds(l * E + off, KB)])

        for r in range(NS // SR):

            @pl.when(jnp.logical_and(s >= r * SR, s < (r + 1) * SR))
            def _stage():
                pltpu.sync_copy(dnl, shd.at[pl.ds((s - r * SR) * NP, NP)])

            plsc.subcore_barrier()
            for t in range(SR):
                pltpu.sync_copy(shd.at[pl.ds(t * NP + s * SLC, SLC)], tbuf)
                if r == 0 and t == 0:

                    @pl.loop(0, SLC // 16)
                    def _init(i):
                        ix = pl.ds(i * 16, 16)
                        red[ix] = tbuf[ix]

                else:

                    @pl.loop(0, SLC // 16)
                    def _acc(i):
                        ix = pl.ds(i * 16, 16)
                        red[ix] = red[ix] + tbuf[ix]

            plsc.subcore_barrier()

        pltpu.sync_copy(red, pden_h.at[pl.ds(c * NP + s * SLC, SLC)])

    return body(src, dst, asf, adf)


def _sc_accum(src, dst, ee, denp, x0, zrows):
    mesh = plsc.VectorSubcoreMesh(core_axis_name="c", subcore_axis_name="s")

    @functools.partial(
        pl.kernel,
        out_type=jax.ShapeDtypeStruct((NC, NL, N, D), F32),
        mesh=mesh,
        scratch_types=[
            pltpu.VMEM((N,), F32),
            pltpu.VMEM((LCH,), jnp.int32),
            pltpu.VMEM((LCH,), jnp.int32),
            pltpu.VMEM((LCH,), jnp.int32),
            pltpu.VMEM((LCH,), jnp.int32),
            pltpu.VMEM((LCH,), F32),
            pltpu.VMEM((LCH,), F32),
            pltpu.VMEM((LCH,), F32),
            pltpu.VMEM((LCH,), F32),
            pltpu.VMEM((KC, D), F32),
            pltpu.VMEM((KC, D), F32),
            pltpu.VMEM_SHARED((N, D), F32),
            pltpu.SemaphoreType.DMA,
            pltpu.SemaphoreType.DMA,
            pltpu.SemaphoreType.DMA,
            pltpu.SemaphoreType.DMA,
            pltpu.SemaphoreType.DMA,
            pltpu.SemaphoreType.DMA,
        ],
        compiler_params=pltpu.CompilerParams(needs_layout_passes=False),
    )
    def body(src_h, dst_h, ee_h, den_h, x0_h, z_h, accp_h,
             dloc, srcv0, srcv1, dstv0, dstv1, eevc0, eevc1, alph0, alph1,
             rows0, rows1, acc, lsem0, lsem1, gsem0, gsem1,
             ssem0, ssem1):
        c = lax.axis_index("c")
        s = lax.axis_index("s")
        wid = c * NS + s
        base = wid * EPT
        svs = (srcv0, srcv1)
        dvs = (dstv0, dstv1)
        evs = (eevc0, eevc1)
        als = (alph0, alph1)
        rws = (rows0, rows1)
        lsems = (lsem0, lsem1)
        gsems = (gsem0, gsem1)
        ssems = (ssem0, ssem1)

        for l in range(NL):

            def _off(i):
                return base + jnp.minimum(i, NLCH - 1) * LCH

            def lin_descs(b, i):
                off = _off(i)
                return (
                    pltpu.make_async_copy(src_h.at[pl.ds(off, LCH)],
                                          svs[b], lsems[b]),
                    pltpu.make_async_copy(dst_h.at[pl.ds(off, LCH)],
                                          dvs[b], lsems[b]),
                    pltpu.make_async_copy(ee_h.at[pl.ds(l * E + off, LCH)],
                                          evs[b], lsems[b]),
                )

            def issue_lin(b, i):
                for d in lin_descs(b, i):
                    d.start()

            def wait_lin(b, i):
                for d in lin_descs(b, i):
                    d.wait()

            def alpha_all(b):
                @pl.loop(0, LCH // 16)
                def _alpha(g):
                    gx = pl.ds(g * 16, 16)
                    dn = plsc.load_gather(dloc, [dvs[b][gx]])
                    als[b][gx] = evs[b][gx] / dn

            def issue_gather(b, k, r):
                pltpu.async_copy(
                    x0_h.at[svs[b].at[pl.ds(k * KC, KC)]], rws[r], gsems[r])

            def wait_gather(b, k, r):
                pltpu.make_async_copy(
                    x0_h.at[svs[b].at[pl.ds(k * KC, KC)]], rws[r],
                    gsems[r]).wait()

            def scale(b, k, r):
                @pl.loop(0, KC)
                def _scale(e):
                    sp = plsc.load_gather(
                        als[b], [jnp.full((16,), k * KC, jnp.int32) + e])
                    for j in range(D // 16):
                        jx = pl.ds(j * 16, 16)
                        rws[r][e, jx] = rws[r][e, jx] * sp

            def issue_scat(b, k, r):
                pltpu.async_copy(rws[r], acc.at[dvs[b].at[pl.ds(k * KC, KC)]],
                                 ssems[r], add=True)

            def wait_scat(b, k, r):
                pltpu.make_async_copy(
                    rws[r], acc.at[dvs[b].at[pl.ds(k * KC, KC)]],
                    ssems[r]).wait()

            pltpu.sync_copy(den_h.at[pl.ds(l * N, N)], dloc)
            pltpu.sync_copy(z_h.at[pl.ds(0, 624)],
                            acc.at[pl.ds(s * 624, 624)])

            @pl.when(s == NS - 1)
            def _ztail():
                pltpu.sync_copy(z_h.at[pl.ds(0, 16)],
                                acc.at[pl.ds(624 * NS, 16)])

            plsc.subcore_barrier()

            issue_lin(0, 0)
            wait_lin(0, 0)
            alpha_all(0)
            issue_gather(0, 0, 0)
            issue_lin(1, 1)

            def sub(t, j, last_pair):
                b, k, r = j // KPL, j % KPL, j % 2
                wait_gather(b, k, r)
                if j == KPL - 1:
                    wait_lin(1, t * 2 + 1)
                    alpha_all(1)
                if j == KPL + 1:
                    issue_lin(0, t * 2 + 2)
                if j == 2 * KPL - 1:
                    wait_lin(0, t * 2 + 2)
                    alpha_all(0)
                if j > 0:
                    wait_scat((j - 1) // KPL, (j - 1) % KPL, 1 - r)
                nj = j + 1
                if nj < 2 * KPL:
                    issue_gather(nj // KPL, nj % KPL, nj % 2)
                elif not last_pair:
                    issue_gather(0, 0, 0)
                scale(b, k, r)
                issue_scat(b, k, r)

            @pl.loop(0, (NLCH - 1) // 2)
            def _pair(t):
                for j in range(2 * KPL):
                    sub(t, j, False)
                wait_scat(1, KPL - 1, 1)
                issue_lin(1, t * 2 + 3)

            for k in range(KPL):
                r = k % 2
                wait_gather(0, k, r)
                if k > 0:
                    wait_scat(0, k - 1, 1 - r)
                if k < KPL - 1:
                    issue_gather(0, k + 1, 1 - r)
                scale(0, k, r)
                issue_scat(0, k, r)
            wait_scat(0, KPL - 1, (KPL - 1) % 2)
            wait_lin(1, NLCH)

            plsc.subcore_barrier()

            pltpu.sync_copy(acc.at[pl.ds(s * 624, 624)],
                            accp_h.at[c, l, pl.ds(s * 624, 624)])

            @pl.when(s == NS - 1)
            def _dtail():
                pltpu.sync_copy(acc.at[pl.ds(624 * NS, 16)],
                                accp_h.at[c, l, pl.ds(624 * NS, 16)])

            plsc.subcore_barrier()

    return body(src, dst, ee, denp, x0, zrows)


def _tc_prep(x, w1, b1, u128):
    R = 400
    grid = (N // R,)

    def body(x_ref, w_ref, b_ref, u_ref, x0_ref, meta_ref, sexp_ref):
        x0 = jnp.dot(x_ref[...], w_ref[...],
                     preferred_element_type=F32) + b_ref[0]
        m = jnp.dot(x0, u_ref[...], preferred_element_type=F32)
        x0_ref[...] = x0
        meta_ref[...] = m
        sexp_ref[...] = jnp.exp(jnp.maximum(m, 0.2 * m))

    return pl.pallas_call(
        body,
        grid=grid,
        in_specs=[
            pl.BlockSpec((R, D), lambda i: (i, 0)),
            pl.BlockSpec((D, D), lambda i: (0, 0)),
            pl.BlockSpec((1, D), lambda i: (0, 0)),
            pl.BlockSpec((D, D), lambda i: (0, 0)),
        ],
        out_specs=[
            pl.BlockSpec((R, D), lambda i: (i, 0)),
            pl.BlockSpec((R, D), lambda i: (i, 0)),
            pl.BlockSpec((R, D), lambda i: (i, 0)),
        ],
        out_shape=[
            jax.ShapeDtypeStruct((N, D), F32),
            jax.ShapeDtypeStruct((N, D), F32),
            jax.ShapeDtypeStruct((N, D), F32),
        ],
    )(x, w1, b1, u128)


def _tc_final(accp, x0, selfw, gat_W, gat_b, wih_t, whh_t, w2, b2):
    R = 400
    grid = (N // R,)

    def body(a_ref, x0_ref, sw_ref, gw_ref, gb_ref, wih_ref, whh_ref,
             w2_ref, b2_ref, out_ref):
        x0 = x0_ref[...]
        sw = sw_ref[...]
        hs = []
        for l in range(NL):
            acc = a_ref[0, l] + a_ref[1, l]
            msg = acc + sw[:, l:l + 1] * x0
            h_l = jnp.tanh(
                jnp.dot(msg, gw_ref[l], preferred_element_type=F32)
                + gb_ref[l, 0])
            hs.append(h_l)
        h = jnp.zeros((R, HID), F32)
        cc = jnp.zeros((R, HID), F32)
        xx = x0
        for l in range(NL):
            cat = jnp.concatenate([hs[l], xx], axis=-1)
            dn = (((1,), (1,)), ((), ()))
            g = (lax.dot_general(cat, wih_ref[l], dn,
                                 preferred_element_type=F32)
                 + lax.dot_general(h, whh_ref[l], dn,
                                   preferred_element_type=F32))
            gi = jax.nn.sigmoid(g[:, 0:HID])
            gf = jax.nn.sigmoid(g[:, HID:2 * HID])
            gg = jnp.tanh(g[:, 2 * HID:3 * HID])
            go = jax.nn.sigmoid(g[:, 3 * HID:4 * HID])
            cc = gf * cc + gi * gg
            h = go * jnp.tanh(cc)
            xx = h + RES_W * x0
        o = jnp.dot(xx, w2_ref[...], preferred_element_type=F32) + b2_ref[0]
        m = jnp.max(o, axis=-1, keepdims=True)
        lse = jnp.log(jnp.sum(jnp.exp(o - m), axis=-1, keepdims=True))
        out_ref[...] = o - m - lse

    return pl.pallas_call(
        body,
        grid=grid,
        in_specs=[
            pl.BlockSpec((NC, NL, R, D), lambda i: (0, 0, i, 0)),
            pl.BlockSpec((R, D), lambda i: (i, 0)),
            pl.BlockSpec((R, D), lambda i: (i, 0)),
            pl.BlockSpec((NL, D, D), lambda i: (0, 0, 0)),
            pl.BlockSpec((NL, 1, D), lambda i: (0, 0, 0)),
            pl.BlockSpec((NL, 4 * HID, 2 * D), lambda i: (0, 0, 0)),
            pl.BlockSpec((NL, 4 * HID, HID), lambda i: (0, 0, 0)),
            pl.BlockSpec((D, D), lambda i: (0, 0)),
            pl.BlockSpec((1, D), lambda i: (0, 0)),
        ],
        out_specs=pl.BlockSpec((R, D), lambda i: (i, 0)),
        out_shape=jax.ShapeDtypeStruct((N, D), F32),
    )(accp, x0, selfw, gat_W, gat_b, wih_t, whh_t, w2, b2)


def kernel(x, edge_index, lin1_w, lin1_b, gat_W, att_src, att_dst, gat_b,
           lstm_Wih, lstm_Whh, lin2_w, lin2_b):
    src = edge_index[0].astype(jnp.int32)
    dst = edge_index[1].astype(jnp.int32)

    u = jnp.einsum("lio,lo->li", gat_W, att_src)
    v = jnp.einsum("lio,lo->li", gat_W, att_dst)
    u128 = jnp.zeros((D, D), F32)
    u128 = u128.at[:, 0:NL].set(u.T)
    u128 = u128.at[:, NL:2 * NL].set(v.T)
    u128 = u128.at[:, 2 * NL:3 * NL].set(u.T + v.T)

    x0, meta, sexp = _tc_prep(x, lin1_w, lin1_b.reshape(1, D), u128)

    as_ = meta[:, 0:NL].T
    ad_ = meta[:, NL:2 * NL].T
    eeself = sexp[:, 2 * NL:3 * NL].T

    asf = as_.reshape(-1)
    adf = ad_.reshape(-1)

    pden, ee = _sc_denom(src, dst, asf, adf)
    pden = pden.reshape(NC, NP)

    denom = (pden[0, :NL * N] + pden[1, :NL * N]
             + eeself.reshape(-1) + 1e-16)
    denp = jnp.pad(denom, (0, NP - NL * N))

    alpha_self = eeself / denom.reshape(NL, N)
    selfw = jnp.zeros((N, D), F32).at[:, 0:NL].set(alpha_self.T)

    zrows = jnp.zeros((624, D), F32)
    accp = _sc_accum(src, dst, ee, denp, x0, zrows)

    return _tc_final(accp, x0, selfw, gat_W,
                     gat_b.reshape(NL, 1, D), lstm_Wih, lstm_Whh,
                     lin2_w, lin2_b.reshape(1, D))


# speedup vs baseline: 43.6811x; 1.0456x over previous
import functools

import jax
import jax.numpy as jnp
from jax import lax
from jax.experimental import pallas as pl
from jax.experimental.pallas import tpu as pltpu
from jax.experimental.pallas import tpu_sc as plsc

N = 10000
E = 320000
D = 128
NL = 3
HID = 128
RES_W = 0.1
F32 = jnp.float32

NC = 2
NS = 16
NW = NC * NS
EPT = E // NW

NP = 30720
SLC = NP // NS
KB = 2000
NCHB = EPT // KB
KC = 80
LCH = 400
KPL = LCH // KC
NLCH = EPT // LCH
NT = N // NS
SR = 8


def _sc_denom(src, dst, asf, adf):
    mesh = plsc.VectorSubcoreMesh(core_axis_name="c", subcore_axis_name="s")

    @functools.partial(
        pl.kernel,
        out_type=(
            jax.ShapeDtypeStruct((NC * NP,), F32),
            jax.ShapeDtypeStruct((NL * E,), F32),
        ),
        mesh=mesh,
        scratch_types=[
            pltpu.VMEM((NL * N,), F32),
            pltpu.VMEM((NL * N,), F32),
            pltpu.VMEM((NP,), F32),
            pltpu.VMEM((KB,), jnp.int32),
            pltpu.VMEM((KB,), jnp.int32),
            pltpu.VMEM((NL * KB,), F32),
            pltpu.VMEM((SLC,), F32),
            pltpu.VMEM((SLC,), F32),
            pltpu.VMEM_SHARED((SR * NP,), F32),
        ],
        compiler_params=pltpu.CompilerParams(needs_layout_passes=False),
    )
    def body(src_h, dst_h, as_h, ad_h, pden_h, ee_h,
             asl, adl, dnl, srcv, dstv, eev, red, tbuf, shd):
        c = lax.axis_index("c")
        s = lax.axis_index("s")
        wid = c * NS + s
        base = wid * EPT

        @pl.loop(0, NP // 16)
        def _zero(i):
            dnl[pl.ds(i * 16, 16)] = jnp.zeros((16,), F32)

        pltpu.sync_copy(as_h, asl)
        pltpu.sync_copy(ad_h, adl)

        for ch in range(NCHB):
            off = base + ch * KB
            pltpu.sync_copy(src_h.at[pl.ds(off, KB)], srcv)
            pltpu.sync_copy(dst_h.at[pl.ds(off, KB)], dstv)

            @pl.loop(0, KB // 16)
            def _edges(g):
                sv = srcv[pl.ds(g * 16, 16)]
                dv = dstv[pl.ds(g * 16, 16)]
                for l in range(NL):
                    a = plsc.load_gather(asl, [sv + l * N])
                    b = plsc.load_gather(adl, [dv + l * N])
                    e = a + b
                    e = jnp.maximum(e, 0.2 * e)
                    ee = jnp.exp(e)
                    eev[pl.ds(l * KB + g * 16, 16)] = ee
                    plsc.addupdate_scatter(dnl, [dv + l * N], ee)

            for l in range(NL):
                pltpu.sync_copy(eev.at[pl.ds(l * KB, KB)],
                                ee_h.at[pl.ds(l * E + off, KB)])

        for r in range(NS // SR):

            @pl.when(jnp.logical_and(s >= r * SR, s < (r + 1) * SR))
            def _stage():
                pltpu.sync_copy(dnl, shd.at[pl.ds((s - r * SR) * NP, NP)])

            plsc.subcore_barrier()
            for t in range(SR):
                pltpu.sync_copy(shd.at[pl.ds(t * NP + s * SLC, SLC)], tbuf)
                if r == 0 and t == 0:

                    @pl.loop(0, SLC // 16)
                    def _init(i):
                        ix = pl.ds(i * 16, 16)
                        red[ix] = tbuf[ix]

                else:

                    @pl.loop(0, SLC // 16)
                    def _acc(i):
                        ix = pl.ds(i * 16, 16)
                        red[ix] = red[ix] + tbuf[ix]

            plsc.subcore_barrier()

        pltpu.sync_copy(red, pden_h.at[pl.ds(c * NP + s * SLC, SLC)])

    return body(src, dst, asf, adf)


def _sc_accum(src, dst, ee, denp, x0, zrows):
    mesh = plsc.VectorSubcoreMesh(core_axis_name="c", subcore_axis_name="s")

    @functools.partial(
        pl.kernel,
        out_type=jax.ShapeDtypeStruct((NC, NL, N, D), F32),
        mesh=mesh,
        scratch_types=[
            pltpu.VMEM((N,), F32),
        ] + [pltpu.VMEM((LCH,), jnp.int32) for _ in range(6)]
          + [pltpu.VMEM((LCH,), F32) for _ in range(6)]
          + [pltpu.VMEM((KC, D), F32) for _ in range(3)]
          + [pltpu.VMEM_SHARED((N, D), F32)]
          + [pltpu.SemaphoreType.DMA for _ in range(9)],
        compiler_params=pltpu.CompilerParams(needs_layout_passes=False),
    )
    def body(src_h, dst_h, ee_h, den_h, x0_h, z_h, accp_h,
             dloc, srcv0, srcv1, srcv2, dstv0, dstv1, dstv2,
             eevc0, eevc1, eevc2, alph0, alph1, alph2,
             rows0, rows1, rows2, acc,
             lsem0, lsem1, lsem2, gsem0, gsem1, gsem2,
             ssem0, ssem1, ssem2):
        c = lax.axis_index("c")
        s = lax.axis_index("s")
        wid = c * NS + s
        base = wid * EPT
        svs = (srcv0, srcv1, srcv2)
        dvs = (dstv0, dstv1, dstv2)
        evs = (eevc0, eevc1, eevc2)
        als = (alph0, alph1, alph2)
        rws = (rows0, rows1, rows2)
        lsems = (lsem0, lsem1, lsem2)
        gsems = (gsem0, gsem1, gsem2)
        ssems = (ssem0, ssem1, ssem2)

        for l in range(NL):

            def _off(i):
                return base + jnp.minimum(i, NLCH - 1) * LCH

            def lin_descs(b, i):
                off = _off(i)
                return (
                    pltpu.make_async_copy(src_h.at[pl.ds(off, LCH)],
                                          svs[b], lsems[b]),
                    pltpu.make_async_copy(dst_h.at[pl.ds(off, LCH)],
                                          dvs[b], lsems[b]),
                    pltpu.make_async_copy(ee_h.at[pl.ds(l * E + off, LCH)],
                                          evs[b], lsems[b]),
                )

            def issue_lin(b, i):
                for d in lin_descs(b, i):
                    d.start()

            def wait_lin(b, i):
                for d in lin_descs(b, i):
                    d.wait()

            def alpha_all(b):
                @pl.loop(0, LCH // 16)
                def _alpha(g):
                    gx = pl.ds(g * 16, 16)
                    dn = plsc.load_gather(dloc, [dvs[b][gx]])
                    als[b][gx] = evs[b][gx] / dn

            def issue_gather(b, k, r):
                pltpu.async_copy(
                    x0_h.at[svs[b].at[pl.ds(k * KC, KC)]], rws[r], gsems[r])

            def wait_gather(b, k, r):
                pltpu.make_async_copy(
                    x0_h.at[svs[b].at[pl.ds(k * KC, KC)]], rws[r],
                    gsems[r]).wait()

            def scale(b, k, r):
                @pl.loop(0, KC)
                def _scale(e):
                    sp = plsc.load_gather(
                        als[b], [jnp.full((16,), k * KC, jnp.int32) + e])
                    for j in range(D // 16):
                        jx = pl.ds(j * 16, 16)
                        rws[r][e, jx] = rws[r][e, jx] * sp

            def issue_scat(b, k, r):
                pltpu.async_copy(rws[r], acc.at[dvs[b].at[pl.ds(k * KC, KC)]],
                                 ssems[r], add=True)

            def wait_scat(b, k, r):
                pltpu.make_async_copy(
                    rws[r], acc.at[dvs[b].at[pl.ds(k * KC, KC)]],
                    ssems[r]).wait()

            pltpu.sync_copy(den_h.at[pl.ds(l * N, N)], dloc)
            pltpu.sync_copy(z_h.at[pl.ds(0, 624)],
                            acc.at[pl.ds(s * 624, 624)])

            @pl.when(s == NS - 1)
            def _ztail():
                pltpu.sync_copy(z_h.at[pl.ds(0, 16)],
                                acc.at[pl.ds(624 * NS, 16)])

            plsc.subcore_barrier()

            issue_lin(0, 0)
            wait_lin(0, 0)
            alpha_all(0)
            issue_gather(0, 0, 0)
            issue_lin(1, 1)

            NB = (NLCH - 1) // 3

            def emit_body(t, first):
                for j in range(3 * KPL):
                    b, k, r = j // KPL, j % KPL, j % 3
                    wait_gather(b, k, r)
                    if j == KPL - 1:
                        wait_lin(1, t * 3 + 1)
                        alpha_all(1)
                    if j == 2 * KPL - 1:
                        wait_lin(2, t * 3 + 2)
                        alpha_all(2)
                    if j == 3 * KPL - 1:
                        wait_lin(0, t * 3 + 3)
                        alpha_all(0)
                    if j >= 2:
                        wait_scat((j - 2) // KPL, (j - 2) % KPL, (j - 2) % 3)
                    elif not first:
                        wait_scat(2, 3 + j, (13 + j) % 3)
                    if j == 1:
                        issue_lin(2, t * 3 + 2)
                    if j == KPL + 1:
                        issue_lin(0, t * 3 + 3)
                    if j == 2 * KPL + 1:
                        issue_lin(1, t * 3 + 4)
                    nj = j + 1
                    if nj < 3 * KPL:
                        issue_gather(nj // KPL, nj % KPL, nj % 3)
                    else:
                        issue_gather(0, 0, 0)
                    scale(b, k, r)
                    issue_scat(b, k, r)

            emit_body(0, True)

            @pl.loop(1, NB)
            def _body(t):
                emit_body(t, False)

            for k in range(KPL):
                r = k % 3
                wait_gather(0, k, r)
                if k >= 2:
                    wait_scat(0, k - 2, (k - 2) % 3)
                else:
                    wait_scat(2, 3 + k, (13 + k) % 3)
                if k < KPL - 1:
                    issue_gather(0, k + 1, (k + 1) % 3)
                scale(0, k, r)
                issue_scat(0, k, r)
            wait_scat(0, KPL - 2, (KPL - 2) % 3)
            wait_scat(0, KPL - 1, (KPL - 1) % 3)
            wait_lin(1, NLCH)

            plsc.subcore_barrier()

            pltpu.sync_copy(acc.at[pl.ds(s * 624, 624)],
                            accp_h.at[c, l, pl.ds(s * 624, 624)])

            @pl.when(s == NS - 1)
            def _dtail():
                pltpu.sync_copy(acc.at[pl.ds(624 * NS, 16)],
                                accp_h.at[c, l, pl.ds(624 * NS, 16)])

            plsc.subcore_barrier()

    return body(src, dst, ee, denp, x0, zrows)


def _tc_prep(x, w1, b1, u128):
    R = 400
    grid = (N // R,)

    def body(x_ref, w_ref, b_ref, u_ref, x0_ref, meta_ref, sexp_ref):
        x0 = jnp.dot(x_ref[...], w_ref[...],
                     preferred_element_type=F32) + b_ref[0]
        m = jnp.dot(x0, u_ref[...], preferred_element_type=F32)
        x0_ref[...] = x0
        meta_ref[...] = m
        sexp_ref[...] = jnp.exp(jnp.maximum(m, 0.2 * m))

    return pl.pallas_call(
        body,
        grid=grid,
        in_specs=[
            pl.BlockSpec((R, D), lambda i: (i, 0)),
            pl.BlockSpec((D, D), lambda i: (0, 0)),
            pl.BlockSpec((1, D), lambda i: (0, 0)),
            pl.BlockSpec((D, D), lambda i: (0, 0)),
        ],
        out_specs=[
            pl.BlockSpec((R, D), lambda i: (i, 0)),
            pl.BlockSpec((R, D), lambda i: (i, 0)),
            pl.BlockSpec((R, D), lambda i: (i, 0)),
        ],
        out_shape=[
            jax.ShapeDtypeStruct((N, D), F32),
            jax.ShapeDtypeStruct((N, D), F32),
            jax.ShapeDtypeStruct((N, D), F32),
        ],
    )(x, w1, b1, u128)


def _tc_final(accp, x0, selfw, gat_W, gat_b, wih_t, whh_t, w2, b2):
    R = 400
    grid = (N // R,)

    def body(a_ref, x0_ref, sw_ref, gw_ref, gb_ref, wih_ref, whh_ref,
             w2_ref, b2_ref, out_ref):
        x0 = x0_ref[...]
        sw = sw_ref[...]
        hs = []
        for l in range(NL):
            acc = a_ref[0, l] + a_ref[1, l]
            msg = acc + sw[:, l:l + 1] * x0
            h_l = jnp.tanh(
                jnp.dot(msg, gw_ref[l], preferred_element_type=F32)
                + gb_ref[l, 0])
            hs.append(h_l)
        h = jnp.zeros((R, HID), F32)
        cc = jnp.zeros((R, HID), F32)
        xx = x0
        for l in range(NL):
            cat = jnp.concatenate([hs[l], xx], axis=-1)
            dn = (((1,), (1,)), ((), ()))
            g = (lax.dot_general(cat, wih_ref[l], dn,
                                 preferred_element_type=F32)
                 + lax.dot_general(h, whh_ref[l], dn,
                                   preferred_element_type=F32))
            gi = jax.nn.sigmoid(g[:, 0:HID])
            gf = jax.nn.sigmoid(g[:, HID:2 * HID])
            gg = jnp.tanh(g[:, 2 * HID:3 * HID])
            go = jax.nn.sigmoid(g[:, 3 * HID:4 * HID])
            cc = gf * cc + gi * gg
            h = go * jnp.tanh(cc)
            xx = h + RES_W * x0
        o = jnp.dot(xx, w2_ref[...], preferred_element_type=F32) + b2_ref[0]
        m = jnp.max(o, axis=-1, keepdims=True)
        lse = jnp.log(jnp.sum(jnp.exp(o - m), axis=-1, keepdims=True))
        out_ref[...] = o - m - lse

    return pl.pallas_call(
        body,
        grid=grid,
        in_specs=[
            pl.BlockSpec((NC, NL, R, D), lambda i: (0, 0, i, 0)),
            pl.BlockSpec((R, D), lambda i: (i, 0)),
            pl.BlockSpec((R, D), lambda i: (i, 0)),
            pl.BlockSpec((NL, D, D), lambda i: (0, 0, 0)),
            pl.BlockSpec((NL, 1, D), lambda i: (0, 0, 0)),
            pl.BlockSpec((NL, 4 * HID, 2 * D), lambda i: (0, 0, 0)),
            pl.BlockSpec((NL, 4 * HID, HID), lambda i: (0, 0, 0)),
            pl.BlockSpec((D, D), lambda i: (0, 0)),
            pl.BlockSpec((1, D), lambda i: (0, 0)),
        ],
        out_specs=pl.BlockSpec((R, D), lambda i: (i, 0)),
        out_shape=jax.ShapeDtypeStruct((N, D), F32),
    )(accp, x0, selfw, gat_W, gat_b, wih_t, whh_t, w2, b2)


def kernel(x, edge_index, lin1_w, lin1_b, gat_W, att_src, att_dst, gat_b,
           lstm_Wih, lstm_Whh, lin2_w, lin2_b):
    src = edge_index[0].astype(jnp.int32)
    dst = edge_index[1].astype(jnp.int32)

    u = jnp.einsum("lio,lo->li", gat_W, att_src)
    v = jnp.einsum("lio,lo->li", gat_W, att_dst)
    u128 = jnp.zeros((D, D), F32)
    u128 = u128.at[:, 0:NL].set(u.T)
    u128 = u128.at[:, NL:2 * NL].set(v.T)
    u128 = u128.at[:, 2 * NL:3 * NL].set(u.T + v.T)

    x0, meta, sexp = _tc_prep(x, lin1_w, lin1_b.reshape(1, D), u128)

    as_ = meta[:, 0:NL].T
    ad_ = meta[:, NL:2 * NL].T
    eeself = sexp[:, 2 * NL:3 * NL].T

    asf = as_.reshape(-1)
    adf = ad_.reshape(-1)

    pden, ee = _sc_denom(src, dst, asf, adf)
    pden = pden.reshape(NC, NP)

    denom = (pden[0, :NL * N] + pden[1, :NL * N]
             + eeself.reshape(-1) + 1e-16)
    denp = jnp.pad(denom, (0, NP - NL * N))

    alpha_self = eeself / denom.reshape(NL, N)
    selfw = jnp.zeros((N, D), F32).at[:, 0:NL].set(alpha_self.T)

    zrows = jnp.zeros((624, D), F32)
    accp = _sc_accum(src, dst, ee, denp, x0, zrows)

    return _tc_final(accp, x0, selfw, gat_W,
                     gat_b.reshape(NL, 1, D), lstm_Wih, lstm_Whh,
                     lin2_w, lin2_b.reshape(1, D))


# speedup vs baseline: 44.0214x; 1.0078x over previous
import functools

import jax
import jax.numpy as jnp
from jax import lax
from jax.experimental import pallas as pl
from jax.experimental.pallas import tpu as pltpu
from jax.experimental.pallas import tpu_sc as plsc

N = 10000
E = 320000
D = 128
NL = 3
HID = 128
RES_W = 0.1
F32 = jnp.float32

NC = 2
NS = 16
NW = NC * NS
EPT = E // NW

NP = 30720
SLC = NP // NS
KB = 2000
NCHB = EPT // KB
KC = 80
LCH = 400
KPL = LCH // KC
NLCH = EPT // LCH
NT = N // NS
SR = 8


def _sc_denom(src, dst, asf, adf):
    mesh = plsc.VectorSubcoreMesh(core_axis_name="c", subcore_axis_name="s")

    @functools.partial(
        pl.kernel,
        out_type=(
            jax.ShapeDtypeStruct((NC * NP,), F32),
            jax.ShapeDtypeStruct((NL * E,), F32),
        ),
        mesh=mesh,
        scratch_types=[
            pltpu.VMEM((NL * N,), F32),
            pltpu.VMEM((NL * N,), F32),
            pltpu.VMEM((NP,), F32),
            pltpu.VMEM((KB,), jnp.int32),
            pltpu.VMEM((KB,), jnp.int32),
            pltpu.VMEM((NL * KB,), F32),
            pltpu.VMEM((SLC,), F32),
            pltpu.VMEM((SLC,), F32),
            pltpu.VMEM_SHARED((SR * NP,), F32),
        ],
        compiler_params=pltpu.CompilerParams(needs_layout_passes=False),
    )
    def body(src_h, dst_h, as_h, ad_h, pden_h, ee_h,
             asl, adl, dnl, srcv, dstv, eev, red, tbuf, shd):
        c = lax.axis_index("c")
        s = lax.axis_index("s")
        wid = c * NS + s
        base = wid * EPT

        @pl.loop(0, NP // 16, unroll=8)
        def _zero(i):
            dnl[pl.ds(i * 16, 16)] = jnp.zeros((16,), F32)

        pltpu.sync_copy(as_h, asl)
        pltpu.sync_copy(ad_h, adl)

        for ch in range(NCHB):
            off = base + ch * KB
            pltpu.sync_copy(src_h.at[pl.ds(off, KB)], srcv)
            pltpu.sync_copy(dst_h.at[pl.ds(off, KB)], dstv)

            @pl.loop(0, KB // 16)
            def _edges(g):
                sv = srcv[pl.ds(g * 16, 16)]
                dv = dstv[pl.ds(g * 16, 16)]
                for l in range(NL):
                    a = plsc.load_gather(asl, [sv + l * N])
                    b = plsc.load_gather(adl, [dv + l * N])
                    e = a + b
                    e = jnp.maximum(e, 0.2 * e)
                    ee = jnp.exp(e)
                    eev[pl.ds(l * KB + g * 16, 16)] = ee
                    plsc.addupdate_scatter(dnl, [dv + l * N], ee)

            for l in range(NL):
                pltpu.sync_copy(eev.at[pl.ds(l * KB, KB)],
                                ee_h.at[pl.ds(l * E + off, KB)])

        for r in range(NS // SR):

            @pl.when(jnp.logical_and(s >= r * SR, s < (r + 1) * SR))
            def _stage():
                pltpu.sync_copy(dnl, shd.at[pl.ds((s - r * SR) * NP, NP)])

            plsc.subcore_barrier()
            for t in range(SR):
                pltpu.sync_copy(shd.at[pl.ds(t * NP + s * SLC, SLC)], tbuf)
                if r == 0 and t == 0:

                    @pl.loop(0, SLC // 16, unroll=8)
                    def _init(i):
                        ix = pl.ds(i * 16, 16)
                        red[ix] = tbuf[ix]

                else:

                    @pl.loop(0, SLC // 16, unroll=8)
                    def _acc(i):
                        ix = pl.ds(i * 16, 16)
                        red[ix] = red[ix] + tbuf[ix]

            plsc.subcore_barrier()

        pltpu.sync_copy(red, pden_h.at[pl.ds(c * NP + s * SLC, SLC)])

    return body(src, dst, asf, adf)


def _sc_accum(src, dst, ee, denp, x0, zrows):
    mesh = plsc.VectorSubcoreMesh(core_axis_name="c", subcore_axis_name="s")

    @functools.partial(
        pl.kernel,
        out_type=jax.ShapeDtypeStruct((NC, NL, N, D), F32),
        mesh=mesh,
        scratch_types=[
            pltpu.VMEM((N,), F32),
        ] + [pltpu.VMEM((LCH,), jnp.int32) for _ in range(6)]
          + [pltpu.VMEM((LCH,), F32) for _ in range(6)]
          + [pltpu.VMEM((KC, D), F32) for _ in range(3)]
          + [pltpu.VMEM_SHARED((N, D), F32)]
          + [pltpu.SemaphoreType.DMA for _ in range(9)],
        compiler_params=pltpu.CompilerParams(needs_layout_passes=False),
    )
    def body(src_h, dst_h, ee_h, den_h, x0_h, z_h, accp_h,
             dloc, srcv0, srcv1, srcv2, dstv0, dstv1, dstv2,
             eevc0, eevc1, eevc2, alph0, alph1, alph2,
             rows0, rows1, rows2, acc,
             lsem0, lsem1, lsem2, gsem0, gsem1, gsem2,
             ssem0, ssem1, ssem2):
        c = lax.axis_index("c")
        s = lax.axis_index("s")
        wid = c * NS + s
        base = wid * EPT
        svs = (srcv0, srcv1, srcv2)
        dvs = (dstv0, dstv1, dstv2)
        evs = (eevc0, eevc1, eevc2)
        als = (alph0, alph1, alph2)
        rws = (rows0, rows1, rows2)
        lsems = (lsem0, lsem1, lsem2)
        gsems = (gsem0, gsem1, gsem2)
        ssems = (ssem0, ssem1, ssem2)

        for l in range(NL):

            def _off(i):
                return base + jnp.minimum(i, NLCH - 1) * LCH

            def lin_descs(b, i):
                off = _off(i)
                return (
                    pltpu.make_async_copy(src_h.at[pl.ds(off, LCH)],
                                          svs[b], lsems[b]),
                    pltpu.make_async_copy(dst_h.at[pl.ds(off, LCH)],
                                          dvs[b], lsems[b]),
                    pltpu.make_async_copy(ee_h.at[pl.ds(l * E + off, LCH)],
                                          evs[b], lsems[b]),
                )

            def issue_lin(b, i):
                for d in lin_descs(b, i):
                    d.start()

            def wait_lin(b, i):
                for d in lin_descs(b, i):
                    d.wait()

            def alpha_all(b):
                @pl.loop(0, LCH // 16)
                def _alpha(g):
                    gx = pl.ds(g * 16, 16)
                    dn = plsc.load_gather(dloc, [dvs[b][gx]])
                    als[b][gx] = evs[b][gx] / dn

            def issue_gather(b, k, r):
                pltpu.async_copy(
                    x0_h.at[svs[b].at[pl.ds(k * KC, KC)]], rws[r], gsems[r])

            def wait_gather(b, k, r):
                pltpu.make_async_copy(
                    x0_h.at[svs[b].at[pl.ds(k * KC, KC)]], rws[r],
                    gsems[r]).wait()

            def scale(b, k, r):
                @pl.loop(0, KC)
                def _scale(e):
                    sp = plsc.load_gather(
                        als[b], [jnp.full((16,), k * KC, jnp.int32) + e])
                    for j in range(D // 16):
                        jx = pl.ds(j * 16, 16)
                        rws[r][e, jx] = rws[r][e, jx] * sp

            def issue_scat(b, k, r):
                pltpu.async_copy(rws[r], acc.at[dvs[b].at[pl.ds(k * KC, KC)]],
                                 ssems[r], add=True)

            def wait_scat(b, k, r):
                pltpu.make_async_copy(
                    rws[r], acc.at[dvs[b].at[pl.ds(k * KC, KC)]],
                    ssems[r]).wait()

            pltpu.sync_copy(den_h.at[pl.ds(l * N, N)], dloc)
            pltpu.sync_copy(z_h.at[pl.ds(0, 624)],
                            acc.at[pl.ds(s * 624, 624)])

            @pl.when(s == NS - 1)
            def _ztail():
                pltpu.sync_copy(z_h.at[pl.ds(0, 16)],
                                acc.at[pl.ds(624 * NS, 16)])

            plsc.subcore_barrier()

            issue_lin(0, 0)
            wait_lin(0, 0)
            alpha_all(0)
            issue_gather(0, 0, 0)
            issue_lin(1, 1)

            NB = (NLCH - 1) // 3

            def emit_body(t, first):
                for j in range(3 * KPL):
                    b, k, r = j // KPL, j % KPL, j % 3
                    wait_gather(b, k, r)
                    if j == KPL - 1:
                        wait_lin(1, t * 3 + 1)
                        alpha_all(1)
                    if j == 2 * KPL - 1:
                        wait_lin(2, t * 3 + 2)
                        alpha_all(2)
                    if j == 3 * KPL - 1:
                        wait_lin(0, t * 3 + 3)
                        alpha_all(0)
                    if j >= 2:
                        wait_scat((j - 2) // KPL, (j - 2) % KPL, (j - 2) % 3)
                    elif not first:
                        wait_scat(2, 3 + j, (13 + j) % 3)
                    if j == 1:
                        issue_lin(2, t * 3 + 2)
                    if j == KPL + 1:
                        issue_lin(0, t * 3 + 3)
                    if j == 2 * KPL + 1:
                        issue_lin(1, t * 3 + 4)
                    nj = j + 1
                    if nj < 3 * KPL:
                        issue_gather(nj // KPL, nj % KPL, nj % 3)
                    else:
                        issue_gather(0, 0, 0)
                    scale(b, k, r)
                    issue_scat(b, k, r)

            emit_body(0, True)

            @pl.loop(1, NB)
            def _body(t):
                emit_body(t, False)

            for k in range(KPL):
                r = k % 3
                wait_gather(0, k, r)
                if k >= 2:
                    wait_scat(0, k - 2, (k - 2) % 3)
                else:
                    wait_scat(2, 3 + k, (13 + k) % 3)
                if k < KPL - 1:
                    issue_gather(0, k + 1, (k + 1) % 3)
                scale(0, k, r)
                issue_scat(0, k, r)
            wait_scat(0, KPL - 2, (KPL - 2) % 3)
            wait_scat(0, KPL - 1, (KPL - 1) % 3)
            wait_lin(1, NLCH)

            plsc.subcore_barrier()

            pltpu.sync_copy(acc.at[pl.ds(s * 624, 624)],
                            accp_h.at[c, l, pl.ds(s * 624, 624)])

            @pl.when(s == NS - 1)
            def _dtail():
                pltpu.sync_copy(acc.at[pl.ds(624 * NS, 16)],
                                accp_h.at[c, l, pl.ds(624 * NS, 16)])

            plsc.subcore_barrier()

    return body(src, dst, ee, denp, x0, zrows)


def _tc_prep(x, w1, b1, u128):
    R = 400
    grid = (N // R,)

    def body(x_ref, w_ref, b_ref, u_ref, x0_ref, meta_ref, sexp_ref):
        x0 = jnp.dot(x_ref[...], w_ref[...],
                     preferred_element_type=F32) + b_ref[0]
        m = jnp.dot(x0, u_ref[...], preferred_element_type=F32)
        x0_ref[...] = x0
        meta_ref[...] = m
        sexp_ref[...] = jnp.exp(jnp.maximum(m, 0.2 * m))

    return pl.pallas_call(
        body,
        grid=grid,
        in_specs=[
            pl.BlockSpec((R, D), lambda i: (i, 0)),
            pl.BlockSpec((D, D), lambda i: (0, 0)),
            pl.BlockSpec((1, D), lambda i: (0, 0)),
            pl.BlockSpec((D, D), lambda i: (0, 0)),
        ],
        out_specs=[
            pl.BlockSpec((R, D), lambda i: (i, 0)),
            pl.BlockSpec((R, D), lambda i: (i, 0)),
            pl.BlockSpec((R, D), lambda i: (i, 0)),
        ],
        out_shape=[
            jax.ShapeDtypeStruct((N, D), F32),
            jax.ShapeDtypeStruct((N, D), F32),
            jax.ShapeDtypeStruct((N, D), F32),
        ],
    )(x, w1, b1, u128)


def _tc_final(accp, x0, selfw, gat_W, gat_b, wih_t, whh_t, w2, b2):
    R = 400
    grid = (N // R,)

    def body(a_ref, x0_ref, sw_ref, gw_ref, gb_ref, wih_ref, whh_ref,
             w2_ref, b2_ref, out_ref):
        x0 = x0_ref[...]
        sw = sw_ref[...]
        hs = []
        for l in range(NL):
            acc = a_ref[0, l] + a_ref[1, l]
            msg = acc + sw[:, l:l + 1] * x0
            h_l = jnp.tanh(
                jnp.dot(msg, gw_ref[l], preferred_element_type=F32)
                + gb_ref[l, 0])
            hs.append(h_l)
        h = jnp.zeros((R, HID), F32)
        cc = jnp.zeros((R, HID), F32)
        xx = x0
        for l in range(NL):
            cat = jnp.concatenate([hs[l], xx], axis=-1)
            dn = (((1,), (1,)), ((), ()))
            g = (lax.dot_general(cat, wih_ref[l], dn,
                                 preferred_element_type=F32)
                 + lax.dot_general(h, whh_ref[l], dn,
                                   preferred_element_type=F32))
            gi = jax.nn.sigmoid(g[:, 0:HID])
            gf = jax.nn.sigmoid(g[:, HID:2 * HID])
            gg = jnp.tanh(g[:, 2 * HID:3 * HID])
            go = jax.nn.sigmoid(g[:, 3 * HID:4 * HID])
            cc = gf * cc + gi * gg
            h = go * jnp.tanh(cc)
            xx = h + RES_W * x0
        o = jnp.dot(xx, w2_ref[...], preferred_element_type=F32) + b2_ref[0]
        m = jnp.max(o, axis=-1, keepdims=True)
        lse = jnp.log(jnp.sum(jnp.exp(o - m), axis=-1, keepdims=True))
        out_ref[...] = o - m - lse

    return pl.pallas_call(
        body,
        grid=grid,
        in_specs=[
            pl.BlockSpec((NC, NL, R, D), lambda i: (0, 0, i, 0)),
            pl.BlockSpec((R, D), lambda i: (i, 0)),
            pl.BlockSpec((R, D), lambda i: (i, 0)),
            pl.BlockSpec((NL, D, D), lambda i: (0, 0, 0)),
            pl.BlockSpec((NL, 1, D), lambda i: (0, 0, 0)),
            pl.BlockSpec((NL, 4 * HID, 2 * D), lambda i: (0, 0, 0)),
            pl.BlockSpec((NL, 4 * HID, HID), lambda i: (0, 0, 0)),
            pl.BlockSpec((D, D), lambda i: (0, 0)),
            pl.BlockSpec((1, D), lambda i: (0, 0)),
        ],
        out_specs=pl.BlockSpec((R, D), lambda i: (i, 0)),
        out_shape=jax.ShapeDtypeStruct((N, D), F32),
    )(accp, x0, selfw, gat_W, gat_b, wih_t, whh_t, w2, b2)


def kernel(x, edge_index, lin1_w, lin1_b, gat_W, att_src, att_dst, gat_b,
           lstm_Wih, lstm_Whh, lin2_w, lin2_b):
    src = edge_index[0].astype(jnp.int32)
    dst = edge_index[1].astype(jnp.int32)

    u = jnp.einsum("lio,lo->li", gat_W, att_src)
    v = jnp.einsum("lio,lo->li", gat_W, att_dst)
    u128 = jnp.zeros((D, D), F32)
    u128 = u128.at[:, 0:NL].set(u.T)
    u128 = u128.at[:, NL:2 * NL].set(v.T)
    u128 = u128.at[:, 2 * NL:3 * NL].set(u.T + v.T)

    x0, meta, sexp = _tc_prep(x, lin1_w, lin1_b.reshape(1, D), u128)

    as_ = meta[:, 0:NL].T
    ad_ = meta[:, NL:2 * NL].T
    eeself = sexp[:, 2 * NL:3 * NL].T

    asf = as_.reshape(-1)
    adf = ad_.reshape(-1)

    pden, ee = _sc_denom(src, dst, asf, adf)
    pden = pden.reshape(NC, NP)

    denom = (pden[0, :NL * N] + pden[1, :NL * N]
             + eeself.reshape(-1) + 1e-16)
    denp = jnp.pad(denom, (0, NP - NL * N))

    alpha_self = eeself / denom.reshape(NL, N)
    selfw = jnp.zeros((N, D), F32).at[:, 0:NL].set(alpha_self.T)

    zrows = jnp.zeros((624, D), F32)
    accp = _sc_accum(src, dst, ee, denp, x0, zrows)

    return _tc_final(accp, x0, selfw, gat_W,
                     gat_b.reshape(NL, 1, D), lstm_Wih, lstm_Whh,
                     lin2_w, lin2_b.reshape(1, D))


# speedup vs baseline: 45.4161x; 1.0317x over previous
import functools

import jax
import jax.numpy as jnp
from jax import lax
from jax.experimental import pallas as pl
from jax.experimental.pallas import tpu as pltpu
from jax.experimental.pallas import tpu_sc as plsc

N = 10000
E = 320000
D = 128
NL = 3
HID = 128
RES_W = 0.1
F32 = jnp.float32

NC = 2
NS = 16
NW = NC * NS
EPT = E // NW

NP = 30720
SLC = NP // NS
KB = 2000
NCHB = EPT // KB
KC = 80
LCH = 400
KPL = LCH // KC
NLCH = EPT // LCH
NT = N // NS
SR = 8


def _sc_denom(src, dst, asf, adf):
    mesh = plsc.VectorSubcoreMesh(core_axis_name="c", subcore_axis_name="s")

    @functools.partial(
        pl.kernel,
        out_type=(
            jax.ShapeDtypeStruct((NC * NP,), F32),
            jax.ShapeDtypeStruct((NL * E,), F32),
        ),
        mesh=mesh,
        scratch_types=[
            pltpu.VMEM((NL * N,), F32),
            pltpu.VMEM((NL * N,), F32),
            pltpu.VMEM((NP,), F32),
            pltpu.VMEM((KB,), jnp.int32),
            pltpu.VMEM((KB,), jnp.int32),
            pltpu.VMEM((NL * KB,), F32),
            pltpu.VMEM((SLC,), F32),
            pltpu.VMEM((SLC,), F32),
            pltpu.VMEM_SHARED((SR * NP,), F32),
        ],
        compiler_params=pltpu.CompilerParams(needs_layout_passes=False),
    )
    def body(src_h, dst_h, as_h, ad_h, pden_h, ee_h,
             asl, adl, dnl, srcv, dstv, eev, red, tbuf, shd):
        c = lax.axis_index("c")
        s = lax.axis_index("s")
        wid = c * NS + s
        base = wid * EPT

        @pl.loop(0, NP // 16, unroll=8)
        def _zero(i):
            dnl[pl.ds(i * 16, 16)] = jnp.zeros((16,), F32)

        pltpu.sync_copy(as_h, asl)
        pltpu.sync_copy(ad_h, adl)

        for ch in range(NCHB):
            off = base + ch * KB
            pltpu.sync_copy(src_h.at[pl.ds(off, KB)], srcv)
            pltpu.sync_copy(dst_h.at[pl.ds(off, KB)], dstv)

            @pl.loop(0, KB // 16)
            def _edges(g):
                sv = srcv[pl.ds(g * 16, 16)]
                dv = dstv[pl.ds(g * 16, 16)]
                for l in range(NL):
                    a = plsc.load_gather(asl, [sv + l * N])
                    b = plsc.load_gather(adl, [dv + l * N])
                    e = a + b
                    e = jnp.maximum(e, 0.2 * e)
                    ee = jnp.exp(e)
                    eev[pl.ds(l * KB + g * 16, 16)] = ee
                    plsc.addupdate_scatter(dnl, [dv + l * N], ee)

            for l in range(NL):
                pltpu.sync_copy(eev.at[pl.ds(l * KB, KB)],
                                ee_h.at[pl.ds(l * E + off, KB)])

        for r in range(NS // SR):

            @pl.when(jnp.logical_and(s >= r * SR, s < (r + 1) * SR))
            def _stage():
                pltpu.sync_copy(dnl, shd.at[pl.ds((s - r * SR) * NP, NP)])

            plsc.subcore_barrier()
            for t in range(SR):
                pltpu.sync_copy(shd.at[pl.ds(t * NP + s * SLC, SLC)], tbuf)
                if r == 0 and t == 0:

                    @pl.loop(0, SLC // 16, unroll=8)
                    def _init(i):
                        ix = pl.ds(i * 16, 16)
                        red[ix] = tbuf[ix]

                else:

                    @pl.loop(0, SLC // 16, unroll=8)
                    def _acc(i):
                        ix = pl.ds(i * 16, 16)
                        red[ix] = red[ix] + tbuf[ix]

            plsc.subcore_barrier()

        pltpu.sync_copy(red, pden_h.at[pl.ds(c * NP + s * SLC, SLC)])

    return body(src, dst, asf, adf)


def _sc_accum(src, dst, ee, denp, x0, zrows):
    mesh = plsc.VectorSubcoreMesh(core_axis_name="c", subcore_axis_name="s")

    @functools.partial(
        pl.kernel,
        out_type=jax.ShapeDtypeStruct((NC, NL, N, D), F32),
        mesh=mesh,
        scratch_types=[
            pltpu.VMEM((N,), F32),
        ] + [pltpu.VMEM((LCH,), jnp.int32) for _ in range(6)]
          + [pltpu.VMEM((LCH,), F32) for _ in range(6)]
          + [pltpu.VMEM((KC, D), F32) for _ in range(3)]
          + [pltpu.VMEM_SHARED((N, D), F32)]
          + [pltpu.SemaphoreType.DMA for _ in range(9)],
        compiler_params=pltpu.CompilerParams(needs_layout_passes=False),
    )
    def body(src_h, dst_h, ee_h, den_h, x0_h, z_h, accp_h,
             dloc, srcv0, srcv1, srcv2, dstv0, dstv1, dstv2,
             eevc0, eevc1, eevc2, alph0, alph1, alph2,
             rows0, rows1, rows2, acc,
             lsem0, lsem1, lsem2, gsem0, gsem1, gsem2,
             ssem0, ssem1, ssem2):
        c = lax.axis_index("c")
        s = lax.axis_index("s")
        wid = c * NS + s
        base = wid * EPT
        svs = (srcv0, srcv1, srcv2)
        dvs = (dstv0, dstv1, dstv2)
        evs = (eevc0, eevc1, eevc2)
        als = (alph0, alph1, alph2)
        rws = (rows0, rows1, rows2)
        lsems = (lsem0, lsem1, lsem2)
        gsems = (gsem0, gsem1, gsem2)
        ssems = (ssem0, ssem1, ssem2)

        for l in range(NL):

            def _off(i):
                return base + jnp.minimum(i, NLCH - 1) * LCH

            def lin_descs(b, i):
                off = _off(i)
                return (
                    pltpu.make_async_copy(src_h.at[pl.ds(off, LCH)],
                                          svs[b], lsems[b]),
                    pltpu.make_async_copy(dst_h.at[pl.ds(off, LCH)],
                                          dvs[b], lsems[b]),
                    pltpu.make_async_copy(ee_h.at[pl.ds(l * E + off, LCH)],
                                          evs[b], lsems[b]),
                )

            def issue_lin(b, i):
                for d in lin_descs(b, i):
                    d.start()

            def wait_lin(b, i):
                for d in lin_descs(b, i):
                    d.wait()

            def alpha_all(b):
                @pl.loop(0, LCH // 16)
                def _alpha(g):
                    gx = pl.ds(g * 16, 16)
                    dn = plsc.load_gather(dloc, [dvs[b][gx]])
                    als[b][gx] = evs[b][gx] / dn

            def issue_gather(b, k, r):
                pltpu.async_copy(
                    x0_h.at[svs[b].at[pl.ds(k * KC, KC)]], rws[r], gsems[r])

            def wait_gather(b, k, r):
                pltpu.make_async_copy(
                    x0_h.at[svs[b].at[pl.ds(k * KC, KC)]], rws[r],
                    gsems[r]).wait()

            def scale(b, k, r):
                @pl.loop(0, KC)
                def _scale(e):
                    sp = plsc.load_gather(
                        als[b], [jnp.full((16,), k * KC, jnp.int32) + e])
                    for j in range(D // 16):
                        jx = pl.ds(j * 16, 16)
                        rws[r][e, jx] = rws[r][e, jx] * sp

            def issue_scat(b, k, r):
                pltpu.async_copy(rws[r], acc.at[dvs[b].at[pl.ds(k * KC, KC)]],
                                 ssems[r], add=True)

            def wait_scat(b, k, r):
                pltpu.make_async_copy(
                    rws[r], acc.at[dvs[b].at[pl.ds(k * KC, KC)]],
                    ssems[r]).wait()

            pltpu.sync_copy(den_h.at[pl.ds(l * N, N)], dloc)
            pltpu.sync_copy(z_h.at[pl.ds(0, 624)],
                            acc.at[pl.ds(s * 624, 624)])

            @pl.when(s == NS - 1)
            def _ztail():
                pltpu.sync_copy(z_h.at[pl.ds(0, 16)],
                                acc.at[pl.ds(624 * NS, 16)])

            plsc.subcore_barrier()

            issue_lin(0, 0)
            wait_lin(0, 0)
            alpha_all(0)
            issue_gather(0, 0, 0)
            issue_lin(1, 1)

            NB = (NLCH - 1) // 3

            def emit_body(t, first):
                for j in range(3 * KPL):
                    b, k, r = j // KPL, j % KPL, j % 3
                    wait_gather(b, k, r)
                    if j == KPL - 1:
                        wait_lin(1, t * 3 + 1)
                        alpha_all(1)
                    if j == 2 * KPL - 1:
                        wait_lin(2, t * 3 + 2)
                        alpha_all(2)
                    if j == 3 * KPL - 1:
                        wait_lin(0, t * 3 + 3)
                        alpha_all(0)
                    if j >= 2:
                        wait_scat((j - 2) // KPL, (j - 2) % KPL, (j - 2) % 3)
                    elif not first:
                        wait_scat(2, 3 + j, (13 + j) % 3)
                    if j == 1:
                        issue_lin(2, t * 3 + 2)
                    if j == KPL + 1:
                        issue_lin(0, t * 3 + 3)
                    if j == 2 * KPL + 1:
                        issue_lin(1, t * 3 + 4)
                    nj = j + 1
                    if nj < 3 * KPL:
                        issue_gather(nj // KPL, nj % KPL, nj % 3)
                    else:
                        issue_gather(0, 0, 0)
                    scale(b, k, r)
                    issue_scat(b, k, r)

            emit_body(0, True)

            @pl.loop(1, NB)
            def _body(t):
                emit_body(t, False)

            for k in range(KPL):
                r = k % 3
                wait_gather(0, k, r)
                if k >= 2:
                    wait_scat(0, k - 2, (k - 2) % 3)
                else:
                    wait_scat(2, 3 + k, (13 + k) % 3)
                if k < KPL - 1:
                    issue_gather(0, k + 1, (k + 1) % 3)
                scale(0, k, r)
                issue_scat(0, k, r)
            wait_scat(0, KPL - 2, (KPL - 2) % 3)
            wait_scat(0, KPL - 1, (KPL - 1) % 3)
            wait_lin(1, NLCH)

            plsc.subcore_barrier()

            pltpu.sync_copy(acc.at[pl.ds(s * 624, 624)],
                            accp_h.at[c, l, pl.ds(s * 624, 624)])

            @pl.when(s == NS - 1)
            def _dtail():
                pltpu.sync_copy(acc.at[pl.ds(624 * NS, 16)],
                                accp_h.at[c, l, pl.ds(624 * NS, 16)])

            plsc.subcore_barrier()

    return body(src, dst, ee, denp, x0, zrows)


def _tc_prep(x, w1, b1, u128):
    R = 1000
    grid = (N // R,)

    def body(x_ref, w_ref, b_ref, u_ref, x0_ref, meta_ref, sexp_ref):
        x0 = jnp.dot(x_ref[...], w_ref[...],
                     preferred_element_type=F32) + b_ref[0]
        m = jnp.dot(x0, u_ref[...], preferred_element_type=F32)
        x0_ref[...] = x0
        meta_ref[...] = m
        sexp_ref[...] = jnp.exp(jnp.maximum(m, 0.2 * m))

    return pl.pallas_call(
        body,
        grid=grid,
        in_specs=[
            pl.BlockSpec((R, D), lambda i: (i, 0)),
            pl.BlockSpec((D, D), lambda i: (0, 0)),
            pl.BlockSpec((1, D), lambda i: (0, 0)),
            pl.BlockSpec((D, D), lambda i: (0, 0)),
        ],
        out_specs=[
            pl.BlockSpec((R, D), lambda i: (i, 0)),
            pl.BlockSpec((R, D), lambda i: (i, 0)),
            pl.BlockSpec((R, D), lambda i: (i, 0)),
        ],
        out_shape=[
            jax.ShapeDtypeStruct((N, D), F32),
            jax.ShapeDtypeStruct((N, D), F32),
            jax.ShapeDtypeStruct((N, D), F32),
        ],
    )(x, w1, b1, u128)


def _tc_final(accp, x0, selfw, gat_W, gat_b, wih_t, whh_t, w2, b2):
    R = 1000
    grid = (N // R,)

    def body(a_ref, x0_ref, sw_ref, gw_ref, gb_ref, wih_ref, whh_ref,
             w2_ref, b2_ref, out_ref):
        x0 = x0_ref[...]
        sw = sw_ref[...]
        hs = []
        for l in range(NL):
            acc = a_ref[0, l] + a_ref[1, l]
            msg = acc + sw[:, l:l + 1] * x0
            h_l = jnp.tanh(
                jnp.dot(msg, gw_ref[l], preferred_element_type=F32)
                + gb_ref[l, 0])
            hs.append(h_l)
        h = jnp.zeros((R, HID), F32)
        cc = jnp.zeros((R, HID), F32)
        xx = x0
        for l in range(NL):
            cat = jnp.concatenate([hs[l], xx], axis=-1)
            dn = (((1,), (1,)), ((), ()))
            g = (lax.dot_general(cat, wih_ref[l], dn,
                                 preferred_element_type=F32)
                 + lax.dot_general(h, whh_ref[l], dn,
                                   preferred_element_type=F32))
            gi = jax.nn.sigmoid(g[:, 0:HID])
            gf = jax.nn.sigmoid(g[:, HID:2 * HID])
            gg = jnp.tanh(g[:, 2 * HID:3 * HID])
            go = jax.nn.sigmoid(g[:, 3 * HID:4 * HID])
            cc = gf * cc + gi * gg
            h = go * jnp.tanh(cc)
            xx = h + RES_W * x0
        o = jnp.dot(xx, w2_ref[...], preferred_element_type=F32) + b2_ref[0]
        m = jnp.max(o, axis=-1, keepdims=True)
        lse = jnp.log(jnp.sum(jnp.exp(o - m), axis=-1, keepdims=True))
        out_ref[...] = o - m - lse

    return pl.pallas_call(
        body,
        grid=grid,
        in_specs=[
            pl.BlockSpec((NC, NL, R, D), lambda i: (0, 0, i, 0)),
            pl.BlockSpec((R, D), lambda i: (i, 0)),
            pl.BlockSpec((R, D), lambda i: (i, 0)),
            pl.BlockSpec((NL, D, D), lambda i: (0, 0, 0)),
            pl.BlockSpec((NL, 1, D), lambda i: (0, 0, 0)),
            pl.BlockSpec((NL, 4 * HID, 2 * D), lambda i: (0, 0, 0)),
            pl.BlockSpec((NL, 4 * HID, HID), lambda i: (0, 0, 0)),
            pl.BlockSpec((D, D), lambda i: (0, 0)),
            pl.BlockSpec((1, D), lambda i: (0, 0)),
        ],
        out_specs=pl.BlockSpec((R, D), lambda i: (i, 0)),
        out_shape=jax.ShapeDtypeStruct((N, D), F32),
    )(accp, x0, selfw, gat_W, gat_b, wih_t, whh_t, w2, b2)


def kernel(x, edge_index, lin1_w, lin1_b, gat_W, att_src, att_dst, gat_b,
           lstm_Wih, lstm_Whh, lin2_w, lin2_b):
    src = edge_index[0].astype(jnp.int32)
    dst = edge_index[1].astype(jnp.int32)

    u = jnp.einsum("lio,lo->li", gat_W, att_src)
    v = jnp.einsum("lio,lo->li", gat_W, att_dst)
    u128 = jnp.zeros((D, D), F32)
    u128 = u128.at[:, 0:NL].set(u.T)
    u128 = u128.at[:, NL:2 * NL].set(v.T)
    u128 = u128.at[:, 2 * NL:3 * NL].set(u.T + v.T)

    x0, meta, sexp = _tc_prep(x, lin1_w, lin1_b.reshape(1, D), u128)

    as_ = meta[:, 0:NL].T
    ad_ = meta[:, NL:2 * NL].T
    eeself = sexp[:, 2 * NL:3 * NL].T

    asf = as_.reshape(-1)
    adf = ad_.reshape(-1)

    pden, ee = _sc_denom(src, dst, asf, adf)
    pden = pden.reshape(NC, NP)

    denom = (pden[0, :NL * N] + pden[1, :NL * N]
             + eeself.reshape(-1) + 1e-16)
    denp = jnp.pad(denom, (0, NP - NL * N))

    alpha_self = eeself / denom.reshape(NL, N)
    selfw = jnp.zeros((N, D), F32).at[:, 0:NL].set(alpha_self.T)

    zrows = jnp.zeros((624, D), F32)
    accp = _sc_accum(src, dst, ee, denp, x0, zrows)

    return _tc_final(accp, x0, selfw, gat_W,
                     gat_b.reshape(NL, 1, D), lstm_Wih, lstm_Whh,
                     lin2_w, lin2_b.reshape(1, D))


# speedup vs baseline: 45.9341x; 1.0114x over previous
import functools

import jax
import jax.numpy as jnp
from jax import lax
from jax.experimental import pallas as pl
from jax.experimental.pallas import tpu as pltpu
from jax.experimental.pallas import tpu_sc as plsc

N = 10000
E = 320000
D = 128
NL = 3
HID = 128
RES_W = 0.1
F32 = jnp.float32

NC = 2
NS = 16
NW = NC * NS
EPT = E // NW

NP = 30720
SLC = NP // NS
KB = 2000
NCHB = EPT // KB
KC = 80
LCH = 400
KPL = LCH // KC
NLCH = EPT // LCH
NT = N // NS
SR = 8


def _sc_denom(src, dst, asf, adf):
    mesh = plsc.VectorSubcoreMesh(core_axis_name="c", subcore_axis_name="s")

    @functools.partial(
        pl.kernel,
        out_type=(
            jax.ShapeDtypeStruct((NC * NP,), F32),
            jax.ShapeDtypeStruct((NL * E,), F32),
        ),
        mesh=mesh,
        scratch_types=[
            pltpu.VMEM((NL * N,), F32),
            pltpu.VMEM((NL * N,), F32),
            pltpu.VMEM((NP,), F32),
            pltpu.VMEM((KB,), jnp.int32),
            pltpu.VMEM((KB,), jnp.int32),
            pltpu.VMEM((NL * KB,), F32),
            pltpu.VMEM((SLC,), F32),
            pltpu.VMEM((SLC,), F32),
            pltpu.VMEM_SHARED((SR * NP,), F32),
        ],
        compiler_params=pltpu.CompilerParams(needs_layout_passes=False),
    )
    def body(src_h, dst_h, as_h, ad_h, pden_h, ee_h,
             asl, adl, dnl, srcv, dstv, eev, red, tbuf, shd):
        c = lax.axis_index("c")
        s = lax.axis_index("s")
        wid = c * NS + s
        base = wid * EPT

        @pl.loop(0, NP // 16, unroll=8)
        def _zero(i):
            dnl[pl.ds(i * 16, 16)] = jnp.zeros((16,), F32)

        pltpu.sync_copy(as_h, asl)
        pltpu.sync_copy(ad_h, adl)

        for ch in range(NCHB):
            off = base + ch * KB
            pltpu.sync_copy(src_h.at[pl.ds(off, KB)], srcv)
            pltpu.sync_copy(dst_h.at[pl.ds(off, KB)], dstv)

            @pl.loop(0, KB // 16)
            def _edges(g):
                sv = srcv[pl.ds(g * 16, 16)]
                dv = dstv[pl.ds(g * 16, 16)]
                for l in range(NL):
                    a = plsc.load_gather(asl, [sv + l * N])
                    b = plsc.load_gather(adl, [dv + l * N])
                    e = a + b
                    e = jnp.maximum(e, 0.2 * e)
                    ee = jnp.exp(e)
                    eev[pl.ds(l * KB + g * 16, 16)] = ee
                    plsc.addupdate_scatter(dnl, [dv + l * N], ee)

            for l in range(NL):
                pltpu.sync_copy(eev.at[pl.ds(l * KB, KB)],
                                ee_h.at[pl.ds(l * E + off, KB)])

        for r in range(NS // SR):

            @pl.when(jnp.logical_and(s >= r * SR, s < (r + 1) * SR))
            def _stage():
                pltpu.sync_copy(dnl, shd.at[pl.ds((s - r * SR) * NP, NP)])

            plsc.subcore_barrier()
            for t in range(SR):
                pltpu.sync_copy(shd.at[pl.ds(t * NP + s * SLC, SLC)], tbuf)
                if r == 0 and t == 0:

                    @pl.loop(0, SLC // 16, unroll=8)
                    def _init(i):
                        ix = pl.ds(i * 16, 16)
                        red[ix] = tbuf[ix]

                else:

                    @pl.loop(0, SLC // 16, unroll=8)
                    def _acc(i):
                        ix = pl.ds(i * 16, 16)
                        red[ix] = red[ix] + tbuf[ix]

            plsc.subcore_barrier()

        pltpu.sync_copy(red, pden_h.at[pl.ds(c * NP + s * SLC, SLC)])

    return body(src, dst, asf, adf)


def _sc_accum(src, dst, ee, denp, x0, zrows):
    mesh = plsc.VectorSubcoreMesh(core_axis_name="c", subcore_axis_name="s")

    @functools.partial(
        pl.kernel,
        out_type=jax.ShapeDtypeStruct((NC, NL, N, D), F32),
        mesh=mesh,
        scratch_types=[
            pltpu.VMEM((N,), F32),
        ] + [pltpu.VMEM((LCH,), jnp.int32) for _ in range(6)]
          + [pltpu.VMEM((LCH,), F32) for _ in range(6)]
          + [pltpu.VMEM((KC, D), F32) for _ in range(3)]
          + [pltpu.VMEM_SHARED((N, D), F32)]
          + [pltpu.SemaphoreType.DMA for _ in range(9)],
        compiler_params=pltpu.CompilerParams(needs_layout_passes=False),
    )
    def body(src_h, dst_h, ee_h, den_h, x0_h, z_h, accp_h,
             dloc, srcv0, srcv1, srcv2, dstv0, dstv1, dstv2,
             eevc0, eevc1, eevc2, alph0, alph1, alph2,
             rows0, rows1, rows2, acc,
             lsem0, lsem1, lsem2, gsem0, gsem1, gsem2,
             ssem0, ssem1, ssem2):
        c = lax.axis_index("c")
        s = lax.axis_index("s")
        wid = c * NS + s
        base = wid * EPT
        svs = (srcv0, srcv1, srcv2)
        dvs = (dstv0, dstv1, dstv2)
        evs = (eevc0, eevc1, eevc2)
        als = (alph0, alph1, alph2)
        rws = (rows0, rows1, rows2)
        lsems = (lsem0, lsem1, lsem2)
        gsems = (gsem0, gsem1, gsem2)
        ssems = (ssem0, ssem1, ssem2)

        for l in range(NL):

            def _off(i):
                return base + jnp.minimum(i, NLCH - 1) * LCH

            def lin_descs(b, i):
                off = _off(i)
                return (
                    pltpu.make_async_copy(src_h.at[pl.ds(off, LCH)],
                                          svs[b], lsems[b]),
                    pltpu.make_async_copy(dst_h.at[pl.ds(off, LCH)],
                                          dvs[b], lsems[b]),
                    pltpu.make_async_copy(ee_h.at[pl.ds(l * E + off, LCH)],
                                          evs[b], lsems[b]),
                )

            def issue_lin(b, i):
                for d in lin_descs(b, i):
                    d.start()

            def wait_lin(b, i):
                for d in lin_descs(b, i):
                    d.wait()

            def alpha_all(b):
                @pl.loop(0, LCH // 16)
                def _alpha(g):
                    gx = pl.ds(g * 16, 16)
                    dn = plsc.load_gather(dloc, [dvs[b][gx]])
                    als[b][gx] = evs[b][gx] / dn

            def issue_gather(b, k, r):
                pltpu.async_copy(
                    x0_h.at[svs[b].at[pl.ds(k * KC, KC)]], rws[r], gsems[r])

            def wait_gather(b, k, r):
                pltpu.make_async_copy(
                    x0_h.at[svs[b].at[pl.ds(k * KC, KC)]], rws[r],
                    gsems[r]).wait()

            def scale(b, k, r):
                @plsc.parallel_loop(0, KC)
                def _scale(e):
                    sp = plsc.load_gather(
                        als[b], [jnp.full((16,), k * KC, jnp.int32) + e])
                    for j in range(D // 16):
                        jx = pl.ds(j * 16, 16)
                        rws[r][e, jx] = rws[r][e, jx] * sp

            def issue_scat(b, k, r):
                pltpu.async_copy(rws[r], acc.at[dvs[b].at[pl.ds(k * KC, KC)]],
                                 ssems[r], add=True)

            def wait_scat(b, k, r):
                pltpu.make_async_copy(
                    rws[r], acc.at[dvs[b].at[pl.ds(k * KC, KC)]],
                    ssems[r]).wait()

            pltpu.sync_copy(den_h.at[pl.ds(l * N, N)], dloc)
            pltpu.sync_copy(z_h.at[pl.ds(0, 624)],
                            acc.at[pl.ds(s * 624, 624)])

            @pl.when(s == NS - 1)
            def _ztail():
                pltpu.sync_copy(z_h.at[pl.ds(0, 16)],
                                acc.at[pl.ds(624 * NS, 16)])

            plsc.subcore_barrier()

            issue_lin(0, 0)
            wait_lin(0, 0)
            alpha_all(0)
            issue_gather(0, 0, 0)
            issue_lin(1, 1)

            NB = (NLCH - 1) // 3

            def emit_body(t, first):
                for j in range(3 * KPL):
                    b, k, r = j // KPL, j % KPL, j % 3
                    wait_gather(b, k, r)
                    if j == KPL - 1:
                        wait_lin(1, t * 3 + 1)
                        alpha_all(1)
                    if j == 2 * KPL - 1:
                        wait_lin(2, t * 3 + 2)
                        alpha_all(2)
                    if j == 3 * KPL - 1:
                        wait_lin(0, t * 3 + 3)
                        alpha_all(0)
                    if j >= 2:
                        wait_scat((j - 2) // KPL, (j - 2) % KPL, (j - 2) % 3)
                    elif not first:
                        wait_scat(2, 3 + j, (13 + j) % 3)
                    if j == 1:
                        issue_lin(2, t * 3 + 2)
                    if j == KPL + 1:
                        issue_lin(0, t * 3 + 3)
                    if j == 2 * KPL + 1:
                        issue_lin(1, t * 3 + 4)
                    nj = j + 1
                    if nj < 3 * KPL:
                        issue_gather(nj // KPL, nj % KPL, nj % 3)
                    else:
                        issue_gather(0, 0, 0)
                    scale(b, k, r)
                    issue_scat(b, k, r)

            emit_body(0, True)

            @pl.loop(1, NB)
            def _body(t):
                emit_body(t, False)

            for k in range(KPL):
                r = k % 3
                wait_gather(0, k, r)
                if k >= 2:
                    wait_scat(0, k - 2, (k - 2) % 3)
                else:
                    wait_scat(2, 3 + k, (13 + k) % 3)
                if k < KPL - 1:
                    issue_gather(0, k + 1, (k + 1) % 3)
                scale(0, k, r)
                issue_scat(0, k, r)
            wait_scat(0, KPL - 2, (KPL - 2) % 3)
            wait_scat(0, KPL - 1, (KPL - 1) % 3)
            wait_lin(1, NLCH)

            plsc.subcore_barrier()

            pltpu.sync_copy(acc.at[pl.ds(s * 624, 624)],
                            accp_h.at[c, l, pl.ds(s * 624, 624)])

            @pl.when(s == NS - 1)
            def _dtail():
                pltpu.sync_copy(acc.at[pl.ds(624 * NS, 16)],
                                accp_h.at[c, l, pl.ds(624 * NS, 16)])

            plsc.subcore_barrier()

    return body(src, dst, ee, denp, x0, zrows)


def _tc_prep(x, w1, b1, u128):
    R = 1000
    grid = (N // R,)

    def body(x_ref, w_ref, b_ref, u_ref, x0_ref, meta_ref, sexp_ref):
        x0 = jnp.dot(x_ref[...], w_ref[...],
                     preferred_element_type=F32) + b_ref[0]
        m = jnp.dot(x0, u_ref[...], preferred_element_type=F32)
        x0_ref[...] = x0
        meta_ref[...] = m
        sexp_ref[...] = jnp.exp(jnp.maximum(m, 0.2 * m))

    return pl.pallas_call(
        body,
        grid=grid,
        in_specs=[
            pl.BlockSpec((R, D), lambda i: (i, 0)),
            pl.BlockSpec((D, D), lambda i: (0, 0)),
            pl.BlockSpec((1, D), lambda i: (0, 0)),
            pl.BlockSpec((D, D), lambda i: (0, 0)),
        ],
        out_specs=[
            pl.BlockSpec((R, D), lambda i: (i, 0)),
            pl.BlockSpec((R, D), lambda i: (i, 0)),
            pl.BlockSpec((R, D), lambda i: (i, 0)),
        ],
        out_shape=[
            jax.ShapeDtypeStruct((N, D), F32),
            jax.ShapeDtypeStruct((N, D), F32),
            jax.ShapeDtypeStruct((N, D), F32),
        ],
    )(x, w1, b1, u128)


def _tc_final(accp, x0, selfw, gat_W, gat_b, wih_t, whh_t, w2, b2):
    R = 1000
    grid = (N // R,)

    def body(a_ref, x0_ref, sw_ref, gw_ref, gb_ref, wih_ref, whh_ref,
             w2_ref, b2_ref, out_ref):
        x0 = x0_ref[...]
        sw = sw_ref[...]
        hs = []
        for l in range(NL):
            acc = a_ref[0, l] + a_ref[1, l]
            msg = acc + sw[:, l:l + 1] * x0
            h_l = jnp.tanh(
                jnp.dot(msg, gw_ref[l], preferred_element_type=F32)
                + gb_ref[l, 0])
            hs.append(h_l)
        h = jnp.zeros((R, HID), F32)
        cc = jnp.zeros((R, HID), F32)
        xx = x0
        for l in range(NL):
            cat = jnp.concatenate([hs[l], xx], axis=-1)
            dn = (((1,), (1,)), ((), ()))
            g = (lax.dot_general(cat, wih_ref[l], dn,
                                 preferred_element_type=F32)
                 + lax.dot_general(h, whh_ref[l], dn,
                                   preferred_element_type=F32))
            gi = jax.nn.sigmoid(g[:, 0:HID])
            gf = jax.nn.sigmoid(g[:, HID:2 * HID])
            gg = jnp.tanh(g[:, 2 * HID:3 * HID])
            go = jax.nn.sigmoid(g[:, 3 * HID:4 * HID])
            cc = gf * cc + gi * gg
            h = go * jnp.tanh(cc)
            xx = h + RES_W * x0
        o = jnp.dot(xx, w2_ref[...], preferred_element_type=F32) + b2_ref[0]
        m = jnp.max(o, axis=-1, keepdims=True)
        lse = jnp.log(jnp.sum(jnp.exp(o - m), axis=-1, keepdims=True))
        out_ref[...] = o - m - lse

    return pl.pallas_call(
        body,
        grid=grid,
        in_specs=[
            pl.BlockSpec((NC, NL, R, D), lambda i: (0, 0, i, 0)),
            pl.BlockSpec((R, D), lambda i: (i, 0)),
            pl.BlockSpec((R, D), lambda i: (i, 0)),
            pl.BlockSpec((NL, D, D), lambda i: (0, 0, 0)),
            pl.BlockSpec((NL, 1, D), lambda i: (0, 0, 0)),
            pl.BlockSpec((NL, 4 * HID, 2 * D), lambda i: (0, 0, 0)),
            pl.BlockSpec((NL, 4 * HID, HID), lambda i: (0, 0, 0)),
            pl.BlockSpec((D, D), lambda i: (0, 0)),
            pl.BlockSpec((1, D), lambda i: (0, 0)),
        ],
        out_specs=pl.BlockSpec((R, D), lambda i: (i, 0)),
        out_shape=jax.ShapeDtypeStruct((N, D), F32),
    )(accp, x0, selfw, gat_W, gat_b, wih_t, whh_t, w2, b2)


def kernel(x, edge_index, lin1_w, lin1_b, gat_W, att_src, att_dst, gat_b,
           lstm_Wih, lstm_Whh, lin2_w, lin2_b):
    src = edge_index[0].astype(jnp.int32)
    dst = edge_index[1].astype(jnp.int32)

    u = jnp.einsum("lio,lo->li", gat_W, att_src)
    v = jnp.einsum("lio,lo->li", gat_W, att_dst)
    u128 = jnp.zeros((D, D), F32)
    u128 = u128.at[:, 0:NL].set(u.T)
    u128 = u128.at[:, NL:2 * NL].set(v.T)
    u128 = u128.at[:, 2 * NL:3 * NL].set(u.T + v.T)

    x0, meta, sexp = _tc_prep(x, lin1_w, lin1_b.reshape(1, D), u128)

    as_ = meta[:, 0:NL].T
    ad_ = meta[:, NL:2 * NL].T
    eeself = sexp[:, 2 * NL:3 * NL].T

    asf = as_.reshape(-1)
    adf = ad_.reshape(-1)

    pden, ee = _sc_denom(src, dst, asf, adf)
    pden = pden.reshape(NC, NP)

    denom = (pden[0, :NL * N] + pden[1, :NL * N]
             + eeself.reshape(-1) + 1e-16)
    denp = jnp.pad(denom, (0, NP - NL * N))

    alpha_self = eeself / denom.reshape(NL, N)
    selfw = jnp.zeros((N, D), F32).at[:, 0:NL].set(alpha_self.T)

    zrows = jnp.zeros((624, D), F32)
    accp = _sc_accum(src, dst, ee, denp, x0, zrows)

    return _tc_final(accp, x0, selfw, gat_W,
                     gat_b.reshape(NL, 1, D), lstm_Wih, lstm_Whh,
                     lin2_w, lin2_b.reshape(1, D))


# speedup vs baseline: 48.5892x; 1.0578x over previous
import functools

import jax
import jax.numpy as jnp
from jax import lax
from jax.experimental import pallas as pl
from jax.experimental.pallas import tpu as pltpu
from jax.experimental.pallas import tpu_sc as plsc

N = 10000
E = 320000
D = 128
NL = 3
HID = 128
RES_W = 0.1
F32 = jnp.float32

NC = 2
NS = 16
NW = NC * NS
EPT = E // NW

NP = 30720
SLC = NP // NS
KB = 2000
NCHB = EPT // KB
KC = 80
LCH = 400
KPL = LCH // KC
NLCH = EPT // LCH
NT = N // NS
SR = 8


def _sc_denom(src, dst, asf, adf):
    mesh = plsc.VectorSubcoreMesh(core_axis_name="c", subcore_axis_name="s")

    @functools.partial(
        pl.kernel,
        out_type=(
            jax.ShapeDtypeStruct((NC * NP,), F32),
            jax.ShapeDtypeStruct((NL * E,), F32),
        ),
        mesh=mesh,
        scratch_types=[
            pltpu.VMEM((NL * N,), F32),
            pltpu.VMEM((NL * N,), F32),
            pltpu.VMEM((NP,), F32),
            pltpu.VMEM((KB,), jnp.int32),
            pltpu.VMEM((KB,), jnp.int32),
            pltpu.VMEM((NL * KB,), F32),
            pltpu.VMEM((SLC,), F32),
            pltpu.VMEM((SLC,), F32),
            pltpu.VMEM_SHARED((SR * NP,), F32),
        ],
        compiler_params=pltpu.CompilerParams(needs_layout_passes=False),
    )
    def body(src_h, dst_h, as_h, ad_h, pden_h, ee_h,
             asl, adl, dnl, srcv, dstv, eev, red, tbuf, shd):
        c = lax.axis_index("c")
        s = lax.axis_index("s")
        wid = c * NS + s
        base = wid * EPT

        @pl.loop(0, NP // 16, unroll=8)
        def _zero(i):
            dnl[pl.ds(i * 16, 16)] = jnp.zeros((16,), F32)

        pltpu.sync_copy(as_h, asl)
        pltpu.sync_copy(ad_h, adl)

        for ch in range(NCHB):
            off = base + ch * KB
            pltpu.sync_copy(src_h.at[pl.ds(off, KB)], srcv)
            pltpu.sync_copy(dst_h.at[pl.ds(off, KB)], dstv)

            @plsc.parallel_loop(0, KB // 16)
            def _edges(g):
                sv = srcv[pl.ds(g * 16, 16)]
                dv = dstv[pl.ds(g * 16, 16)]
                for l in range(NL):
                    a = plsc.load_gather(asl, [sv + l * N])
                    b = plsc.load_gather(adl, [dv + l * N])
                    e = a + b
                    e = jnp.maximum(e, 0.2 * e)
                    ee = jnp.exp(e)
                    eev[pl.ds(l * KB + g * 16, 16)] = ee
                    plsc.addupdate_scatter(dnl, [dv + l * N], ee)

            for l in range(NL):
                pltpu.sync_copy(eev.at[pl.ds(l * KB, KB)],
                                ee_h.at[pl.ds(l * E + off, KB)])

        for r in range(NS // SR):

            @pl.when(jnp.logical_and(s >= r * SR, s < (r + 1) * SR))
            def _stage():
                pltpu.sync_copy(dnl, shd.at[pl.ds((s - r * SR) * NP, NP)])

            plsc.subcore_barrier()
            for t in range(SR):
                pltpu.sync_copy(shd.at[pl.ds(t * NP + s * SLC, SLC)], tbuf)
                if r == 0 and t == 0:

                    @pl.loop(0, SLC // 16, unroll=8)
                    def _init(i):
                        ix = pl.ds(i * 16, 16)
                        red[ix] = tbuf[ix]

                else:

                    @pl.loop(0, SLC // 16, unroll=8)
                    def _acc(i):
                        ix = pl.ds(i * 16, 16)
                        red[ix] = red[ix] + tbuf[ix]

            plsc.subcore_barrier()

        pltpu.sync_copy(red, pden_h.at[pl.ds(c * NP + s * SLC, SLC)])

    return body(src, dst, asf, adf)


def _sc_accum(src, dst, ee, denp, x0, zrows):
    mesh = plsc.VectorSubcoreMesh(core_axis_name="c", subcore_axis_name="s")

    @functools.partial(
        pl.kernel,
        out_type=jax.ShapeDtypeStruct((NC, NL, N, D), F32),
        mesh=mesh,
        scratch_types=[
            pltpu.VMEM((N,), F32),
        ] + [pltpu.VMEM((LCH,), jnp.int32) for _ in range(6)]
          + [pltpu.VMEM((LCH,), F32) for _ in range(6)]
          + [pltpu.VMEM((KC, D), F32) for _ in range(3)]
          + [pltpu.VMEM_SHARED((N, D), F32)]
          + [pltpu.SemaphoreType.DMA for _ in range(9)],
        compiler_params=pltpu.CompilerParams(needs_layout_passes=False),
    )
    def body(src_h, dst_h, ee_h, den_h, x0_h, z_h, accp_h,
             dloc, srcv0, srcv1, srcv2, dstv0, dstv1, dstv2,
             eevc0, eevc1, eevc2, alph0, alph1, alph2,
             rows0, rows1, rows2, acc,
             lsem0, lsem1, lsem2, gsem0, gsem1, gsem2,
             ssem0, ssem1, ssem2):
        c = lax.axis_index("c")
        s = lax.axis_index("s")
        wid = c * NS + s
        base = wid * EPT
        svs = (srcv0, srcv1, srcv2)
        dvs = (dstv0, dstv1, dstv2)
        evs = (eevc0, eevc1, eevc2)
        als = (alph0, alph1, alph2)
        rws = (rows0, rows1, rows2)
        lsems = (lsem0, lsem1, lsem2)
        gsems = (gsem0, gsem1, gsem2)
        ssems = (ssem0, ssem1, ssem2)

        for l in range(NL):

            def _off(i):
                return base + jnp.minimum(i, NLCH - 1) * LCH

            def lin_descs(b, i):
                off = _off(i)
                return (
                    pltpu.make_async_copy(src_h.at[pl.ds(off, LCH)],
                                          svs[b], lsems[b]),
                    pltpu.make_async_copy(dst_h.at[pl.ds(off, LCH)],
                                          dvs[b], lsems[b]),
                    pltpu.make_async_copy(ee_h.at[pl.ds(l * E + off, LCH)],
                                          evs[b], lsems[b]),
                )

            def issue_lin(b, i):
                for d in lin_descs(b, i):
                    d.start()

            def wait_lin(b, i):
                for d in lin_descs(b, i):
                    d.wait()

            def alpha_all(b):
                @plsc.parallel_loop(0, LCH // 16)
                def _alpha(g):
                    gx = pl.ds(g * 16, 16)
                    dn = plsc.load_gather(dloc, [dvs[b][gx]])
                    als[b][gx] = evs[b][gx] / dn

            def issue_gather(b, k, r):
                pltpu.async_copy(
                    x0_h.at[svs[b].at[pl.ds(k * KC, KC)]], rws[r], gsems[r])

            def wait_gather(b, k, r):
                pltpu.make_async_copy(
                    x0_h.at[svs[b].at[pl.ds(k * KC, KC)]], rws[r],
                    gsems[r]).wait()

            def scale(b, k, r):
                @plsc.parallel_loop(0, KC)
                def _scale(e):
                    sp = plsc.load_gather(
                        als[b], [jnp.full((16,), k * KC, jnp.int32) + e])
                    for j in range(D // 16):
                        jx = pl.ds(j * 16, 16)
                        rws[r][e, jx] = rws[r][e, jx] * sp

            def issue_scat(b, k, r):
                pltpu.async_copy(rws[r], acc.at[dvs[b].at[pl.ds(k * KC, KC)]],
                                 ssems[r], add=True)

            def wait_scat(b, k, r):
                pltpu.make_async_copy(
                    rws[r], acc.at[dvs[b].at[pl.ds(k * KC, KC)]],
                    ssems[r]).wait()

            pltpu.sync_copy(den_h.at[pl.ds(l * N, N)], dloc)
            pltpu.sync_copy(z_h.at[pl.ds(0, 624)],
                            acc.at[pl.ds(s * 624, 624)])

            @pl.when(s == NS - 1)
            def _ztail():
                pltpu.sync_copy(z_h.at[pl.ds(0, 16)],
                                acc.at[pl.ds(624 * NS, 16)])

            plsc.subcore_barrier()

            issue_lin(0, 0)
            wait_lin(0, 0)
            alpha_all(0)
            issue_gather(0, 0, 0)
            issue_lin(1, 1)

            NB = (NLCH - 1) // 3

            def emit_body(t, first):
                for j in range(3 * KPL):
                    b, k, r = j // KPL, j % KPL, j % 3
                    wait_gather(b, k, r)
                    if j == KPL - 1:
                        wait_lin(1, t * 3 + 1)
                        alpha_all(1)
                    if j == 2 * KPL - 1:
                        wait_lin(2, t * 3 + 2)
                        alpha_all(2)
                    if j == 3 * KPL - 1:
                        wait_lin(0, t * 3 + 3)
                        alpha_all(0)
                    if j >= 2:
                        wait_scat((j - 2) // KPL, (j - 2) % KPL, (j - 2) % 3)
                    elif not first:
                        wait_scat(2, 3 + j, (13 + j) % 3)
                    if j == 1:
                        issue_lin(2, t * 3 + 2)
                    if j == KPL + 1:
                        issue_lin(0, t * 3 + 3)
                    if j == 2 * KPL + 1:
                        issue_lin(1, t * 3 + 4)
                    nj = j + 1
                    if nj < 3 * KPL:
                        issue_gather(nj // KPL, nj % KPL, nj % 3)
                    else:
                        issue_gather(0, 0, 0)
                    scale(b, k, r)
                    issue_scat(b, k, r)

            emit_body(0, True)

            @pl.loop(1, NB)
            def _body(t):
                emit_body(t, False)

            for k in range(KPL):
                r = k % 3
                wait_gather(0, k, r)
                if k >= 2:
                    wait_scat(0, k - 2, (k - 2) % 3)
                else:
                    wait_scat(2, 3 + k, (13 + k) % 3)
                if k < KPL - 1:
                    issue_gather(0, k + 1, (k + 1) % 3)
                scale(0, k, r)
                issue_scat(0, k, r)
            wait_scat(0, KPL - 2, (KPL - 2) % 3)
            wait_scat(0, KPL - 1, (KPL - 1) % 3)
            wait_lin(1, NLCH)

            plsc.subcore_barrier()

            pltpu.sync_copy(acc.at[pl.ds(s * 624, 624)],
                            accp_h.at[c, l, pl.ds(s * 624, 624)])

            @pl.when(s == NS - 1)
            def _dtail():
                pltpu.sync_copy(acc.at[pl.ds(624 * NS, 16)],
                                accp_h.at[c, l, pl.ds(624 * NS, 16)])

            plsc.subcore_barrier()

    return body(src, dst, ee, denp, x0, zrows)


def _tc_prep(x, w1, b1, u128):
    R = 1000
    grid = (N // R,)

    def body(x_ref, w_ref, b_ref, u_ref, x0_ref, meta_ref, sexp_ref):
        x0 = jnp.dot(x_ref[...], w_ref[...],
                     preferred_element_type=F32) + b_ref[0]
        m = jnp.dot(x0, u_ref[...], preferred_element_type=F32)
        x0_ref[...] = x0
        meta_ref[...] = m
        sexp_ref[...] = jnp.exp(jnp.maximum(m, 0.2 * m))

    return pl.pallas_call(
        body,
        grid=grid,
        in_specs=[
            pl.BlockSpec((R, D), lambda i: (i, 0)),
            pl.BlockSpec((D, D), lambda i: (0, 0)),
            pl.BlockSpec((1, D), lambda i: (0, 0)),
            pl.BlockSpec((D, D), lambda i: (0, 0)),
        ],
        out_specs=[
            pl.BlockSpec((R, D), lambda i: (i, 0)),
            pl.BlockSpec((R, D), lambda i: (i, 0)),
            pl.BlockSpec((R, D), lambda i: (i, 0)),
        ],
        out_shape=[
            jax.ShapeDtypeStruct((N, D), F32),
            jax.ShapeDtypeStruct((N, D), F32),
            jax.ShapeDtypeStruct((N, D), F32),
        ],
    )(x, w1, b1, u128)


def _tc_final(accp, x0, selfw, gat_W, gat_b, wih_t, whh_t, w2, b2):
    R = 1000
    grid = (N // R,)

    def body(a_ref, x0_ref, sw_ref, gw_ref, gb_ref, wih_ref, whh_ref,
             w2_ref, b2_ref, out_ref):
        x0 = x0_ref[...]
        sw = sw_ref[...]
        hs = []
        for l in range(NL):
            acc = a_ref[0, l] + a_ref[1, l]
            msg = acc + sw[:, l:l + 1] * x0
            h_l = jnp.tanh(
                jnp.dot(msg, gw_ref[l], preferred_element_type=F32)
                + gb_ref[l, 0])
            hs.append(h_l)
        h = jnp.zeros((R, HID), F32)
        cc = jnp.zeros((R, HID), F32)
        xx = x0
        for l in range(NL):
            cat = jnp.concatenate([hs[l], xx], axis=-1)
            dn = (((1,), (1,)), ((), ()))
            g = (lax.dot_general(cat, wih_ref[l], dn,
                                 preferred_element_type=F32)
                 + lax.dot_general(h, whh_ref[l], dn,
                                   preferred_element_type=F32))
            gi = jax.nn.sigmoid(g[:, 0:HID])
            gf = jax.nn.sigmoid(g[:, HID:2 * HID])
            gg = jnp.tanh(g[:, 2 * HID:3 * HID])
            go = jax.nn.sigmoid(g[:, 3 * HID:4 * HID])
            cc = gf * cc + gi * gg
            h = go * jnp.tanh(cc)
            xx = h + RES_W * x0
        o = jnp.dot(xx, w2_ref[...], preferred_element_type=F32) + b2_ref[0]
        m = jnp.max(o, axis=-1, keepdims=True)
        lse = jnp.log(jnp.sum(jnp.exp(o - m), axis=-1, keepdims=True))
        out_ref[...] = o - m - lse

    return pl.pallas_call(
        body,
        grid=grid,
        in_specs=[
            pl.BlockSpec((NC, NL, R, D), lambda i: (0, 0, i, 0)),
            pl.BlockSpec((R, D), lambda i: (i, 0)),
            pl.BlockSpec((R, D), lambda i: (i, 0)),
            pl.BlockSpec((NL, D, D), lambda i: (0, 0, 0)),
            pl.BlockSpec((NL, 1, D), lambda i: (0, 0, 0)),
            pl.BlockSpec((NL, 4 * HID, 2 * D), lambda i: (0, 0, 0)),
            pl.BlockSpec((NL, 4 * HID, HID), lambda i: (0, 0, 0)),
            pl.BlockSpec((D, D), lambda i: (0, 0)),
            pl.BlockSpec((1, D), lambda i: (0, 0)),
        ],
        out_specs=pl.BlockSpec((R, D), lambda i: (i, 0)),
        out_shape=jax.ShapeDtypeStruct((N, D), F32),
    )(accp, x0, selfw, gat_W, gat_b, wih_t, whh_t, w2, b2)


def kernel(x, edge_index, lin1_w, lin1_b, gat_W, att_src, att_dst, gat_b,
           lstm_Wih, lstm_Whh, lin2_w, lin2_b):
    src = edge_index[0].astype(jnp.int32)
    dst = edge_index[1].astype(jnp.int32)

    u = jnp.einsum("lio,lo->li", gat_W, att_src)
    v = jnp.einsum("lio,lo->li", gat_W, att_dst)
    u128 = jnp.zeros((D, D), F32)
    u128 = u128.at[:, 0:NL].set(u.T)
    u128 = u128.at[:, NL:2 * NL].set(v.T)
    u128 = u128.at[:, 2 * NL:3 * NL].set(u.T + v.T)

    x0, meta, sexp = _tc_prep(x, lin1_w, lin1_b.reshape(1, D), u128)

    as_ = meta[:, 0:NL].T
    ad_ = meta[:, NL:2 * NL].T
    eeself = sexp[:, 2 * NL:3 * NL].T

    asf = as_.reshape(-1)
    adf = ad_.reshape(-1)

    pden, ee = _sc_denom(src, dst, asf, adf)
    pden = pden.reshape(NC, NP)

    denom = (pden[0, :NL * N] + pden[1, :NL * N]
             + eeself.reshape(-1) + 1e-16)
    denp = jnp.pad(denom, (0, NP - NL * N))

    alpha_self = eeself / denom.reshape(NL, N)
    selfw = jnp.zeros((N, D), F32).at[:, 0:NL].set(alpha_self.T)

    zrows = jnp.zeros((624, D), F32)
    accp = _sc_accum(src, dst, ee, denp, x0, zrows)

    return _tc_final(accp, x0, selfw, gat_W,
                     gat_b.reshape(NL, 1, D), lstm_Wih, lstm_Whh,
                     lin2_w, lin2_b.reshape(1, D))
